# Initial kernel scaffold; baseline (speedup 1.0000x reference)
import jax
import jax.numpy as jnp
from jax.experimental import pallas as pl


def kernel(x, edge_index, edge_weight, W1, b1, W2, b2):
    raise NotImplementedError("write your pallas kernel here")


# speedup vs baseline: 3.0836x; 3.0836x over previous
import functools

import jax
import jax.numpy as jnp
from jax.experimental import pallas as pl
from jax.experimental.pallas import tpu as pltpu

N_NODES = 10000
N_EDGES = 320000
D_IN = 128
H = 64
K = 16
KM_ITERS = 10

_INIT_IDX = (0, 666, 1333, 1999, 2666, 3333, 3999, 4666,
             5332, 5999, 6666, 7332, 7999, 8665, 9332, 9999)

_INTERPRET = False


_BLK = 500
_NBLK = N_NODES // _BLK


def _prep_body(x_ref, w1_ref, degs_ref, hs_ref, dinv_ref):
    w1 = w1_ref[...]

    def blk(i, _):
        rows = pl.ds(i * _BLK, _BLK)
        s = degs_ref[0, rows, :] + degs_ref[1, rows, :]
        dinv = jax.lax.rsqrt(1.0 + s[:, 0:2])
        h = jnp.dot(x_ref[rows, :], w1, preferred_element_type=jnp.float32)
        hs_ref[rows, :] = h * dinv[:, 0:1]
        dinv_ref[rows, :] = dinv
        return 0

    jax.lax.fori_loop(0, _NBLK, blk, 0)


def _prep_call(x, w1, deg_partials):
    return pl.pallas_call(
        _prep_body,
        out_shape=(
            jax.ShapeDtypeStruct((N_NODES, H), jnp.float32),
            jax.ShapeDtypeStruct((N_NODES, 2), jnp.float32),
        ),
        interpret=_INTERPRET,
    )(x, w1, deg_partials)


def _layer_body(a0_ref, a1_ref, hs_ref, dinv_ref, b_ref, wn_ref, dinvn_ref,
                out_ref, h_ref):
    b = b_ref[...]

    def hblk(i, _):
        rows = pl.ds(i * _BLK, _BLK)
        h_ref[rows, :] = ((a0_ref[rows, :] + a1_ref[rows, :]
                           + hs_ref[rows, :]) * dinv_ref[rows, :] + b)
        return 0

    jax.lax.fori_loop(0, _NBLK, hblk, 0)

    c0 = jnp.concatenate([h_ref[i:i + 1, :] for i in _INIT_IDX], axis=0)

    def stats(c, with_sq):
        cn = jnp.sum(c * c, axis=1)

        def blk(i, carry):
            counts, sums, sqs = carry
            rows = pl.ds(i * _BLK, _BLK)
            hb = h_ref[rows, :]
            g = jax.lax.dot_general(hb, c, (((1,), (1,)), ((), ())),
                                    preferred_element_type=jnp.float32,
                                    precision=jax.lax.Precision.HIGHEST)
            s = cn[None, :] - 2.0 * g
            assign = jnp.argmin(s, axis=1).astype(jnp.int32)
            oh = (assign[None, :] ==
                  jax.lax.broadcasted_iota(jnp.int32, (K, _BLK), 0)
                  ).astype(jnp.float32)
            counts = counts + jnp.sum(oh, axis=1)
            sums = sums + jnp.dot(oh, hb, preferred_element_type=jnp.float32,
                                  precision=jax.lax.Precision.HIGHEST)
            if with_sq:
                sqs = sqs + jnp.dot(oh, hb * hb,
                                    preferred_element_type=jnp.float32,
                                    precision=jax.lax.Precision.HIGHEST)
            return counts, sums, sqs

        z = jnp.zeros((K, H), jnp.float32)
        init = (jnp.zeros((1, K), jnp.float32).reshape(K), z, z)
        return jax.lax.fori_loop(0, _NBLK, blk, init)

    def iter_fn(t, c):
        counts, sums, _ = stats(c, False)
        newc = sums / jnp.maximum(counts, 1.0)[:, None]
        return jnp.where(counts[:, None] > 0, newc, c)

    c = jax.lax.fori_loop(0, KM_ITERS, iter_fn, c0)

    counts, _, sqs = stats(c, True)
    counts = jnp.maximum(counts, 1.0)
    sq = sqs / counts[:, None] - c * c
    vs = jnp.sqrt(jnp.maximum(sq, 1e-4))
    p = 0.5 / (vs * vs)

    const = jnp.sum(c * c * p, axis=1)
    cp2 = 2.0 * c * p
    wn = wn_ref[...]

    def fblk(i, _):
        rows = pl.ds(i * _BLK, _BLK)
        hb = h_ref[rows, :]
        hhb = hb * hb
        t1 = jax.lax.dot_general(hhb, p, (((1,), (1,)), ((), ())),
                                 preferred_element_type=jnp.float32,
                                 precision=jax.lax.Precision.HIGHEST)
        t2 = jax.lax.dot_general(hb, cp2, (((1,), (1,)), ((), ())),
                                 preferred_element_type=jnp.float32,
                                 precision=jax.lax.Precision.HIGHEST)
        logm = t2 - t1 - const[None, :]
        m = jnp.max(logm, axis=1, keepdims=True)
        e = jnp.exp(logm - m)
        frs = e / jnp.sum(e, axis=1, keepdims=True)
        fuzz = jnp.dot(frs, c, preferred_element_type=jnp.float32)
        out_ref[rows, :] = (jnp.dot(fuzz, wn,
                                    preferred_element_type=jnp.float32)
                            * dinvn_ref[rows, :])
        return 0

    jax.lax.fori_loop(0, _NBLK, fblk, 0)


def _layer_call(a0, a1, hs, dinv, b, wn, dinvn):
    return pl.pallas_call(
        _layer_body,
        out_shape=jax.ShapeDtypeStruct((N_NODES, H), jnp.float32),
        scratch_shapes=[pltpu.VMEM((N_NODES, H), jnp.float32)],
        interpret=_INTERPRET,
    )(a0, a1, hs, dinv, b, wn, dinvn)


def _deg_partials(dst, w):
    s1 = jax.ops.segment_sum(w, dst, num_segments=N_NODES)
    s2 = jax.ops.segment_sum(jnp.ones_like(w), dst, num_segments=N_NODES)
    p0 = jnp.zeros((N_NODES, 16), jnp.float32)
    p0 = p0.at[:, 0].set(s1).at[:, 1].set(s2)
    return jnp.stack([p0, jnp.zeros_like(p0)])


def _agg(src, dst, w, rows):
    vals = rows[src] if w is None else w[:, None] * rows[src]
    a = jax.ops.segment_sum(vals, dst, num_segments=N_NODES)
    return a, jnp.zeros_like(a)


def kernel(x, edge_index, edge_weight, W1, b1, W2, b2):
    src = edge_index[0]
    dst = edge_index[1]
    degp = _deg_partials(dst, edge_weight)

    hs1, dinv = _prep_call(x, W1, degp)
    dinv1 = dinv[:, 0:1]
    dinv2 = dinv[:, 1:2]

    a0, a1 = _agg(src, dst, edge_weight, hs1)
    hs2 = _layer_call(a0, a1, hs1, dinv1, b1.reshape(1, H), W2, dinv2)

    b0p, b1p = _agg(src, dst, None, hs2)
    eye = jnp.eye(H, dtype=jnp.float32)
    ones = jnp.ones((N_NODES, 1), jnp.float32)
    out = _layer_call(b0p, b1p, hs2, dinv2, b2.reshape(1, H), eye, ones)
    return out


# speedup vs baseline: 6.2539x; 2.0281x over previous
import functools

import jax
import jax.numpy as jnp
from jax.experimental import pallas as pl
from jax.experimental.pallas import tpu as pltpu

N_NODES = 10000
N_EDGES = 320000
D_IN = 128
H = 64
K = 16
KM_ITERS = 10

_INIT_IDX = (0, 666, 1333, 1999, 2666, 3333, 3999, 4666,
             5332, 5999, 6666, 7332, 7999, 8665, 9332, 9999)

_INTERPRET = False


_BLK = 500
_NBLK = N_NODES // _BLK


def _prep_body(x_ref, w1_ref, degs_ref, hs_ref, dinv_ref):
    w1 = w1_ref[...]

    def blk(i, _):
        rows = pl.ds(i * _BLK, _BLK)
        s = degs_ref[0, rows, :] + degs_ref[1, rows, :]
        dinv = jax.lax.rsqrt(1.0 + s)
        h = jnp.dot(x_ref[rows, :], w1, preferred_element_type=jnp.float32)
        hs_ref[rows, :] = h * dinv[:, 0:1]
        dinv_ref[rows, :] = dinv
        return 0

    jax.lax.fori_loop(0, _NBLK, blk, 0)


def _prep_call(x, w1, deg_partials):
    return pl.pallas_call(
        _prep_body,
        out_shape=(
            jax.ShapeDtypeStruct((N_NODES, H), jnp.float32),
            jax.ShapeDtypeStruct((N_NODES, 2), jnp.float32),
        ),
        interpret=_INTERPRET,
    )(x, w1, deg_partials)


def _layer_body(a0_ref, a1_ref, hs_ref, dinv_ref, b_ref, wn_ref, dinvn_ref,
                out_ref, h_ref):
    b = b_ref[...]

    def hblk(i, _):
        rows = pl.ds(i * _BLK, _BLK)
        h_ref[rows, :] = ((a0_ref[rows, :] + a1_ref[rows, :]
                           + hs_ref[rows, :]) * dinv_ref[rows, :] + b)
        return 0

    jax.lax.fori_loop(0, _NBLK, hblk, 0)

    c0 = jnp.concatenate([h_ref[i:i + 1, :] for i in _INIT_IDX], axis=0)

    def stats(c, with_sq):
        cn = jnp.sum(c * c, axis=1)

        def blk(i, carry):
            counts, sums, sqs = carry
            rows = pl.ds(i * _BLK, _BLK)
            hb = h_ref[rows, :]
            g = jax.lax.dot_general(hb, c, (((1,), (1,)), ((), ())),
                                    preferred_element_type=jnp.float32,
                                    precision=jax.lax.Precision.HIGHEST)
            s = cn[None, :] - 2.0 * g
            assign = jnp.argmin(s, axis=1).astype(jnp.int32)
            oh = (assign[None, :] ==
                  jax.lax.broadcasted_iota(jnp.int32, (K, _BLK), 0)
                  ).astype(jnp.float32)
            counts = counts + jnp.sum(oh, axis=1)
            sums = sums + jnp.dot(oh, hb, preferred_element_type=jnp.float32,
                                  precision=jax.lax.Precision.HIGHEST)
            if with_sq:
                sqs = sqs + jnp.dot(oh, hb * hb,
                                    preferred_element_type=jnp.float32,
                                    precision=jax.lax.Precision.HIGHEST)
            return counts, sums, sqs

        z = jnp.zeros((K, H), jnp.float32)
        init = (jnp.zeros((1, K), jnp.float32).reshape(K), z, z)
        return jax.lax.fori_loop(0, _NBLK, blk, init)

    def iter_fn(t, c):
        counts, sums, _ = stats(c, False)
        newc = sums / jnp.maximum(counts, 1.0)[:, None]
        return jnp.where(counts[:, None] > 0, newc, c)

    c = jax.lax.fori_loop(0, KM_ITERS, iter_fn, c0)

    counts, _, sqs = stats(c, True)
    counts = jnp.maximum(counts, 1.0)
    sq = sqs / counts[:, None] - c * c
    vs = jnp.sqrt(jnp.maximum(sq, 1e-4))
    p = 0.5 / (vs * vs)

    const = jnp.sum(c * c * p, axis=1)
    cp2 = 2.0 * c * p
    wn = wn_ref[...]

    def fblk(i, _):
        rows = pl.ds(i * _BLK, _BLK)
        hb = h_ref[rows, :]
        hhb = hb * hb
        t1 = jax.lax.dot_general(hhb, p, (((1,), (1,)), ((), ())),
                                 preferred_element_type=jnp.float32,
                                 precision=jax.lax.Precision.HIGHEST)
        t2 = jax.lax.dot_general(hb, cp2, (((1,), (1,)), ((), ())),
                                 preferred_element_type=jnp.float32,
                                 precision=jax.lax.Precision.HIGHEST)
        logm = t2 - t1 - const[None, :]
        m = jnp.max(logm, axis=1, keepdims=True)
        e = jnp.exp(logm - m)
        frs = e / jnp.sum(e, axis=1, keepdims=True)
        fuzz = jnp.dot(frs, c, preferred_element_type=jnp.float32)
        out_ref[rows, :] = (jnp.dot(fuzz, wn,
                                    preferred_element_type=jnp.float32)
                            * dinvn_ref[rows, :])
        return 0

    jax.lax.fori_loop(0, _NBLK, fblk, 0)


def _layer_call(a0, a1, hs, dinv, b, wn, dinvn):
    return pl.pallas_call(
        _layer_body,
        out_shape=jax.ShapeDtypeStruct((N_NODES, H), jnp.float32),
        scratch_shapes=[pltpu.VMEM((N_NODES, H), jnp.float32)],
        interpret=_INTERPRET,
    )(a0, a1, hs, dinv, b, wn, dinvn)


from jax import lax
from jax.experimental.pallas import tpu_sc as plsc

_NC = 2
_NS = 16
_NW = _NC * _NS
_EPW = N_EDGES // _NW
_EC = 80
_NCH = _EPW // _EC
_CPT = 10
_RPT = N_NODES // _CPT
_ZR = 200
_ZRF = 2 * N_NODES // _CPT


def _sc_mesh():
    return plsc.VectorSubcoreMesh(core_axis_name="c", subcore_axis_name="s",
                                  num_cores=_NC, num_subcores=_NS)


def _deg_partials(dst, w):
    @functools.partial(
        pl.kernel,
        out_type=jax.ShapeDtypeStruct((_NC * 2 * N_NODES,), jnp.float32),
        mesh=_sc_mesh(),
        scratch_types=[
            pltpu.VMEM((_EC,), jnp.int32),
            pltpu.VMEM((_EC,), jnp.int32),
            pltpu.VMEM((_EC,), jnp.int32),
            pltpu.VMEM((_EC,), jnp.float32),
            pltpu.VMEM((_EC,), jnp.float32),
            pltpu.VMEM((_ZRF,), jnp.float32),
            pltpu.VMEM_SHARED((2 * N_NODES,), jnp.float32),
        ],
    )
    def deg(dst_hbm, w_hbm, out_hbm, didx, de_v, do_v, w_v, one_v, z_v,
            shared):
        cid = lax.axis_index("c")
        sid = lax.axis_index("s")
        wid = sid * _NC + cid

        z16 = jnp.zeros((16,), jnp.float32)
        o16 = jnp.ones((16,), jnp.float32)

        def zi(i, _):
            z_v[pl.ds(i * 16, 16)] = z16
            return 0

        lax.fori_loop(0, _ZRF // 16, zi, 0)
        for g in range(_EC // 16):
            one_v[pl.ds(g * 16, 16)] = o16

        @pl.when(sid < _CPT)
        def _():
            zb = pl.multiple_of(sid * _ZRF, 8)
            pltpu.sync_copy(z_v, shared.at[pl.ds(zb, _ZRF)])

        plsc.subcore_barrier()
        base = wid * _EPW

        def chunk(ch, _):
            eb = pl.multiple_of(base + ch * _EC, 8)
            pltpu.sync_copy(dst_hbm.at[pl.ds(eb, _EC)], didx)
            pltpu.sync_copy(w_hbm.at[pl.ds(eb, _EC)], w_v)
            for g in range(_EC // 16):
                sl = pl.ds(g * 16, 16)
                d2 = didx[sl] * 2
                de_v[sl] = d2
                do_v[sl] = d2 + 1
            pltpu.sync_copy(w_v, shared.at[de_v], add=True)
            pltpu.sync_copy(one_v, shared.at[do_v], add=True)
            return 0

        lax.fori_loop(0, _NCH, chunk, 0)
        plsc.subcore_barrier()

        @pl.when(sid < _CPT)
        def _():
            zb = pl.multiple_of(sid * _ZRF, 8)
            ob = pl.multiple_of(cid * 2 * N_NODES + sid * _ZRF, 8)
            pltpu.sync_copy(shared.at[pl.ds(zb, _ZRF)], z_v)
            pltpu.sync_copy(z_v, out_hbm.at[pl.ds(ob, _ZRF)])

    return deg(dst, w).reshape(_NC, N_NODES, 2)


def _agg(src, dst, w, rows, weighted):
    @functools.partial(
        pl.kernel,
        out_type=jax.ShapeDtypeStruct((_NC, N_NODES, H), jnp.float32),
        mesh=_sc_mesh(),
        scratch_types=[
            pltpu.VMEM((_EC,), jnp.int32),
            pltpu.VMEM((_EC,), jnp.int32),
            pltpu.VMEM((_EC,), jnp.float32),
            pltpu.VMEM((_EC, H), jnp.float32),
            pltpu.VMEM((_ZR, H), jnp.float32),
            pltpu.VMEM_SHARED((N_NODES, H), jnp.float32),
            pltpu.SemaphoreType.DMA,
        ],
        compiler_params=pltpu.CompilerParams(use_tc_tiling_on_sc=False),
    )
    def agg(src_hbm, dst_hbm, w_hbm, rows_hbm, out_hbm,
            sidx, didx, w_v, rows_v, z_v, shared, sem):
        cid = lax.axis_index("c")
        sid = lax.axis_index("s")
        wid = sid * _NC + cid

        z16 = jnp.zeros((16,), jnp.float32)

        def zi(i, _):
            for q in range(H // 16):
                z_v[i, pl.ds(q * 16, 16)] = z16
            return 0

        lax.fori_loop(0, _ZR, zi, 0)

        @pl.when(sid < _CPT)
        def _():
            for j in range(_RPT // _ZR):
                r0 = pl.multiple_of(sid * _RPT + j * _ZR, 8)
                pltpu.sync_copy(z_v, shared.at[pl.ds(r0, _ZR)])

        plsc.subcore_barrier()
        base = wid * _EPW

        def chunk(ch, _):
            eb = pl.multiple_of(base + ch * _EC, 8)
            pltpu.sync_copy(src_hbm.at[pl.ds(eb, _EC)], sidx)
            pltpu.sync_copy(dst_hbm.at[pl.ds(eb, _EC)], didx)
            pltpu.async_copy(rows_hbm.at[sidx], rows_v, sem).wait()
            if weighted:
                pltpu.sync_copy(w_hbm.at[pl.ds(eb, _EC)], w_v)
                for g in range(_EC // 16):
                    w16 = w_v[pl.ds(g * 16, 16)]
                    for e in range(16):
                        sp = w16.at[jnp.full((16,), e, jnp.int32)].get(
                            mode="promise_in_bounds")
                        r = g * 16 + e
                        for q in range(H // 16):
                            sl = pl.ds(q * 16, 16)
                            rows_v[r, sl] = rows_v[r, sl] * sp
            pltpu.sync_copy(rows_v, shared.at[didx], add=True)
            return 0

        lax.fori_loop(0, _NCH, chunk, 0)
        plsc.subcore_barrier()

        @pl.when(sid < _CPT)
        def _():
            for j in range(_RPT // _ZR):
                r0 = pl.multiple_of(sid * _RPT + j * _ZR, 8)
                pltpu.sync_copy(shared.at[pl.ds(r0, _ZR)], z_v)
                pltpu.sync_copy(z_v, out_hbm.at[cid, pl.ds(r0, _ZR)])

    a = agg(src, dst, w, rows)
    return a[0], a[1]


def kernel(x, edge_index, edge_weight, W1, b1, W2, b2):
    src = edge_index[0]
    dst = edge_index[1]
    degp = _deg_partials(dst, edge_weight)

    hs1, dinv = _prep_call(x, W1, degp)
    dinv1 = dinv[:, 0:1]
    dinv2 = dinv[:, 1:2]

    a0, a1 = _agg(src, dst, edge_weight, hs1, True)
    hs2 = _layer_call(a0, a1, hs1, dinv1, b1.reshape(1, H), W2, dinv2)

    b0p, b1p = _agg(src, dst, edge_weight, hs2, False)
    eye = jnp.eye(H, dtype=jnp.float32)
    ones = jnp.ones((N_NODES, 1), jnp.float32)
    out = _layer_call(b0p, b1p, hs2, dinv2, b2.reshape(1, H), eye, ones)
    return out


# speedup vs baseline: 12.6261x; 2.0189x over previous
import functools

import jax
import jax.numpy as jnp
from jax.experimental import pallas as pl
from jax.experimental.pallas import tpu as pltpu

N_NODES = 10000
N_EDGES = 320000
D_IN = 128
H = 64
K = 16
KM_ITERS = 10

_INIT_IDX = (0, 666, 1333, 1999, 2666, 3333, 3999, 4666,
             5332, 5999, 6666, 7332, 7999, 8665, 9332, 9999)

_INTERPRET = False


_BLK = 500
_NBLK = N_NODES // _BLK


def _prep_body(x_ref, w1_ref, degs_ref, hs_ref, dinv_ref):
    w1 = w1_ref[...]

    def blk(i, _):
        rows = pl.ds(i * _BLK, _BLK)
        s = degs_ref[0, rows, :] + degs_ref[1, rows, :]
        dinv = jax.lax.rsqrt(1.0 + s)
        h = jnp.dot(x_ref[rows, :], w1, preferred_element_type=jnp.float32)
        hs_ref[rows, :] = h * dinv[:, 0:1]
        dinv_ref[rows, :] = dinv
        return 0

    jax.lax.fori_loop(0, _NBLK, blk, 0)


def _prep_call(x, w1, deg_partials):
    return pl.pallas_call(
        _prep_body,
        out_shape=(
            jax.ShapeDtypeStruct((N_NODES, H), jnp.float32),
            jax.ShapeDtypeStruct((N_NODES, 2), jnp.float32),
        ),
        interpret=_INTERPRET,
    )(x, w1, deg_partials)


def _layer_body(a0_ref, a1_ref, hs_ref, dinv_ref, b_ref, wn_ref, dinvn_ref,
                out_ref, h_ref):
    b = b_ref[...]

    def hblk(i, _):
        rows = pl.ds(i * _BLK, _BLK)
        h_ref[rows, :] = ((a0_ref[rows, :] + a1_ref[rows, :]
                           + hs_ref[rows, :]) * dinv_ref[rows, :] + b)
        return 0

    jax.lax.fori_loop(0, _NBLK, hblk, 0)

    c0 = jnp.concatenate([h_ref[i:i + 1, :] for i in _INIT_IDX], axis=0)

    ones_b = jnp.ones((_BLK, 1), jnp.float32)

    def stats(c, with_sq):
        cn = jnp.sum(c * c, axis=1)

        def blk(i, carry):
            counts, sums, sqs = carry
            rows = pl.ds(i * _BLK, _BLK)
            hb = h_ref[rows, :]
            g = jax.lax.dot_general(c, hb, (((1,), (1,)), ((), ())),
                                    preferred_element_type=jnp.float32,
                                    precision=jax.lax.Precision.HIGHEST)
            s = cn[:, None] - 2.0 * g
            mn = jnp.min(s, axis=0, keepdims=True)
            rowi = jax.lax.broadcasted_iota(
                jnp.int32, (K, _BLK), 0).astype(jnp.float32)
            masked = jnp.where(s == mn, rowi, jnp.float32(K))
            amin = jnp.min(masked, axis=0, keepdims=True)
            oh = (rowi == amin).astype(jnp.float32)
            counts = counts + jnp.dot(
                oh, ones_b, preferred_element_type=jnp.float32,
                precision=jax.lax.Precision.HIGHEST)
            sums = sums + jnp.dot(oh, hb, preferred_element_type=jnp.float32,
                                  precision=jax.lax.Precision.HIGHEST)
            if with_sq:
                sqs = sqs + jnp.dot(oh, hb * hb,
                                    preferred_element_type=jnp.float32,
                                    precision=jax.lax.Precision.HIGHEST)
            return counts, sums, sqs

        z = jnp.zeros((K, H), jnp.float32)
        init = (jnp.zeros((K, 1), jnp.float32), z, z)
        counts, sums, sqs = jax.lax.fori_loop(0, _NBLK, blk, init)
        return counts.reshape(K), sums, sqs

    def iter_fn(t, c):
        counts, sums, _ = stats(c, False)
        newc = sums / jnp.maximum(counts, 1.0)[:, None]
        return jnp.where(counts[:, None] > 0, newc, c)

    c = jax.lax.fori_loop(0, KM_ITERS, iter_fn, c0)

    counts, _, sqs = stats(c, True)
    counts = jnp.maximum(counts, 1.0)
    sq = sqs / counts[:, None] - c * c
    vs = jnp.sqrt(jnp.maximum(sq, 1e-4))
    p = 0.5 / (vs * vs)

    const = jnp.sum(c * c * p, axis=1)
    cp2 = 2.0 * c * p
    wn = wn_ref[...]

    def fblk(i, _):
        rows = pl.ds(i * _BLK, _BLK)
        hb = h_ref[rows, :]
        hhb = hb * hb
        t1 = jax.lax.dot_general(p, hhb, (((1,), (1,)), ((), ())),
                                 preferred_element_type=jnp.float32,
                                 precision=jax.lax.Precision.HIGHEST)
        t2 = jax.lax.dot_general(cp2, hb, (((1,), (1,)), ((), ())),
                                 preferred_element_type=jnp.float32,
                                 precision=jax.lax.Precision.HIGHEST)
        logm = t2 - t1 - const[:, None]
        m = jnp.max(logm, axis=0, keepdims=True)
        e = jnp.exp(logm - m)
        frs = e / jnp.sum(e, axis=0, keepdims=True)
        fuzz = jax.lax.dot_general(frs, c, (((0,), (0,)), ((), ())),
                                   preferred_element_type=jnp.float32)
        out_ref[rows, :] = (jnp.dot(fuzz, wn,
                                    preferred_element_type=jnp.float32)
                            * dinvn_ref[rows, :])
        return 0

    jax.lax.fori_loop(0, _NBLK, fblk, 0)


def _layer_call(a0, a1, hs, dinv, b, wn, dinvn):
    return pl.pallas_call(
        _layer_body,
        out_shape=jax.ShapeDtypeStruct((N_NODES, H), jnp.float32),
        scratch_shapes=[pltpu.VMEM((N_NODES, H), jnp.float32)],
        interpret=_INTERPRET,
    )(a0, a1, hs, dinv, b, wn, dinvn)


from jax import lax
from jax.experimental.pallas import tpu_sc as plsc

_NC = 2
_NS = 16
_NW = _NC * _NS
_EPW = N_EDGES // _NW
_EC = 80
_NCH = _EPW // _EC
_CPT = 10
_RPT = N_NODES // _CPT
_ZR = 200
_ZRF = 2 * N_NODES // _CPT


def _sc_mesh():
    return plsc.VectorSubcoreMesh(core_axis_name="c", subcore_axis_name="s",
                                  num_cores=_NC, num_subcores=_NS)


def _deg_partials(dst, w):
    @functools.partial(
        pl.kernel,
        out_type=jax.ShapeDtypeStruct((_NC * 2 * N_NODES,), jnp.float32),
        mesh=_sc_mesh(),
        scratch_types=[
            pltpu.VMEM((_EC,), jnp.int32),
            pltpu.VMEM((_EC,), jnp.int32),
            pltpu.VMEM((_EC,), jnp.int32),
            pltpu.VMEM((_EC,), jnp.float32),
            pltpu.VMEM((_EC,), jnp.float32),
            pltpu.VMEM((_ZRF,), jnp.float32),
            pltpu.VMEM_SHARED((2 * N_NODES,), jnp.float32),
        ],
    )
    def deg(dst_hbm, w_hbm, out_hbm, didx, de_v, do_v, w_v, one_v, z_v,
            shared):
        cid = lax.axis_index("c")
        sid = lax.axis_index("s")
        wid = sid * _NC + cid

        z16 = jnp.zeros((16,), jnp.float32)
        o16 = jnp.ones((16,), jnp.float32)

        def zi(i, _):
            z_v[pl.ds(i * 16, 16)] = z16
            return 0

        lax.fori_loop(0, _ZRF // 16, zi, 0)
        for g in range(_EC // 16):
            one_v[pl.ds(g * 16, 16)] = o16

        @pl.when(sid < _CPT)
        def _():
            zb = pl.multiple_of(sid * _ZRF, 8)
            pltpu.sync_copy(z_v, shared.at[pl.ds(zb, _ZRF)])

        plsc.subcore_barrier()
        base = wid * _EPW

        def chunk(ch, _):
            eb = pl.multiple_of(base + ch * _EC, 8)
            pltpu.sync_copy(dst_hbm.at[pl.ds(eb, _EC)], didx)
            pltpu.sync_copy(w_hbm.at[pl.ds(eb, _EC)], w_v)
            for g in range(_EC // 16):
                sl = pl.ds(g * 16, 16)
                d2 = didx[sl] * 2
                de_v[sl] = d2
                do_v[sl] = d2 + 1
            pltpu.sync_copy(w_v, shared.at[de_v], add=True)
            pltpu.sync_copy(one_v, shared.at[do_v], add=True)
            return 0

        lax.fori_loop(0, _NCH, chunk, 0)
        plsc.subcore_barrier()

        @pl.when(sid < _CPT)
        def _():
            zb = pl.multiple_of(sid * _ZRF, 8)
            ob = pl.multiple_of(cid * 2 * N_NODES + sid * _ZRF, 8)
            pltpu.sync_copy(shared.at[pl.ds(zb, _ZRF)], z_v)
            pltpu.sync_copy(z_v, out_hbm.at[pl.ds(ob, _ZRF)])

    return deg(dst, w).reshape(_NC, N_NODES, 2)


def _agg(src, dst, w, rows, weighted):
    @functools.partial(
        pl.kernel,
        out_type=jax.ShapeDtypeStruct((_NC, N_NODES, H), jnp.float32),
        mesh=_sc_mesh(),
        scratch_types=[
            pltpu.VMEM((_EC,), jnp.int32),
            pltpu.VMEM((_EC,), jnp.int32),
            pltpu.VMEM((_EC,), jnp.float32),
            pltpu.VMEM((_EC, H), jnp.float32),
            pltpu.VMEM((_ZR, H), jnp.float32),
            pltpu.VMEM_SHARED((N_NODES, H), jnp.float32),
            pltpu.SemaphoreType.DMA,
        ],
        compiler_params=pltpu.CompilerParams(use_tc_tiling_on_sc=False),
    )
    def agg(src_hbm, dst_hbm, w_hbm, rows_hbm, out_hbm,
            sidx, didx, w_v, rows_v, z_v, shared, sem):
        cid = lax.axis_index("c")
        sid = lax.axis_index("s")
        wid = sid * _NC + cid

        z16 = jnp.zeros((16,), jnp.float32)

        def zi(i, _):
            for q in range(H // 16):
                z_v[i, pl.ds(q * 16, 16)] = z16
            return 0

        lax.fori_loop(0, _ZR, zi, 0)

        @pl.when(sid < _CPT)
        def _():
            for j in range(_RPT // _ZR):
                r0 = pl.multiple_of(sid * _RPT + j * _ZR, 8)
                pltpu.sync_copy(z_v, shared.at[pl.ds(r0, _ZR)])

        plsc.subcore_barrier()
        base = wid * _EPW

        def chunk(ch, _):
            eb = pl.multiple_of(base + ch * _EC, 8)
            pltpu.sync_copy(src_hbm.at[pl.ds(eb, _EC)], sidx)
            pltpu.sync_copy(dst_hbm.at[pl.ds(eb, _EC)], didx)
            pltpu.async_copy(rows_hbm.at[sidx], rows_v, sem).wait()
            if weighted:
                pltpu.sync_copy(w_hbm.at[pl.ds(eb, _EC)], w_v)
                for g in range(_EC // 16):
                    w16 = w_v[pl.ds(g * 16, 16)]
                    for e in range(16):
                        sp = w16.at[jnp.full((16,), e, jnp.int32)].get(
                            mode="promise_in_bounds")
                        r = g * 16 + e
                        for q in range(H // 16):
                            sl = pl.ds(q * 16, 16)
                            rows_v[r, sl] = rows_v[r, sl] * sp
            pltpu.sync_copy(rows_v, shared.at[didx], add=True)
            return 0

        lax.fori_loop(0, _NCH, chunk, 0)
        plsc.subcore_barrier()

        @pl.when(sid < _CPT)
        def _():
            for j in range(_RPT // _ZR):
                r0 = pl.multiple_of(sid * _RPT + j * _ZR, 8)
                pltpu.sync_copy(shared.at[pl.ds(r0, _ZR)], z_v)
                pltpu.sync_copy(z_v, out_hbm.at[cid, pl.ds(r0, _ZR)])

    a = agg(src, dst, w, rows)
    return a[0], a[1]


def kernel(x, edge_index, edge_weight, W1, b1, W2, b2):
    src = edge_index[0]
    dst = edge_index[1]
    degp = _deg_partials(dst, edge_weight)

    hs1, dinv = _prep_call(x, W1, degp)
    dinv1 = dinv[:, 0:1]
    dinv2 = dinv[:, 1:2]

    a0, a1 = _agg(src, dst, edge_weight, hs1, True)
    hs2 = _layer_call(a0, a1, hs1, dinv1, b1.reshape(1, H), W2, dinv2)

    b0p, b1p = _agg(src, dst, edge_weight, hs2, False)
    eye = jnp.eye(H, dtype=jnp.float32)
    ones = jnp.ones((N_NODES, 1), jnp.float32)
    out = _layer_call(b0p, b1p, hs2, dinv2, b2.reshape(1, H), eye, ones)
    return out


# speedup vs baseline: 20.7286x; 1.6417x over previous
import functools

import jax
import jax.numpy as jnp
from jax.experimental import pallas as pl
from jax.experimental.pallas import tpu as pltpu

N_NODES = 10000
N_EDGES = 320000
D_IN = 128
H = 64
K = 16
KM_ITERS = 10

_INIT_IDX = (0, 666, 1333, 1999, 2666, 3333, 3999, 4666,
             5332, 5999, 6666, 7332, 7999, 8665, 9332, 9999)

_INTERPRET = False


_BLK = 500
_NBLK = N_NODES // _BLK


def _prep_body(x_ref, w1_ref, degs_ref, hs_ref, dinv_ref):
    w1 = w1_ref[...]

    def blk(i, _):
        rows = pl.ds(i * _BLK, _BLK)
        s = degs_ref[0, rows, :] + degs_ref[1, rows, :]
        dinv = jax.lax.rsqrt(1.0 + s)
        h = jnp.dot(x_ref[rows, :], w1, preferred_element_type=jnp.float32)
        hs_ref[rows, :] = h * dinv[:, 0:1]
        dinv_ref[rows, :] = dinv
        return 0

    jax.lax.fori_loop(0, _NBLK, blk, 0)


def _prep_call(x, w1, deg_partials):
    return pl.pallas_call(
        _prep_body,
        out_shape=(
            jax.ShapeDtypeStruct((N_NODES, H), jnp.float32),
            jax.ShapeDtypeStruct((N_NODES, 2), jnp.float32),
        ),
        interpret=_INTERPRET,
    )(x, w1, deg_partials)


def _layer_body(a0_ref, a1_ref, hs_ref, dinv_ref, b_ref, wn_ref, dinvn_ref,
                out_ref, h_ref):
    b = b_ref[...]

    def hblk(i, _):
        rows = pl.ds(i * _BLK, _BLK)
        h_ref[rows, :] = ((a0_ref[rows, :] + a1_ref[rows, :]
                           + hs_ref[rows, :]) * dinv_ref[rows, :] + b)
        return 0

    jax.lax.fori_loop(0, _NBLK, hblk, 0)

    c0 = jnp.concatenate([h_ref[i:i + 1, :] for i in _INIT_IDX], axis=0)

    ones_b = jnp.ones((_BLK, 1), jnp.float32)

    def stats(c, with_sq):
        cn = jnp.sum(c * c, axis=1)

        def blk(i, carry):
            counts, sums, sqs = carry
            rows = pl.ds(i * _BLK, _BLK)
            hb = h_ref[rows, :]
            g = jax.lax.dot_general(c, hb, (((1,), (1,)), ((), ())),
                                    preferred_element_type=jnp.float32,
                                    precision=jax.lax.Precision.HIGHEST)
            s = cn[:, None] - 2.0 * g
            mn = jnp.min(s, axis=0, keepdims=True)
            rowi = jax.lax.broadcasted_iota(
                jnp.int32, (K, _BLK), 0).astype(jnp.float32)
            masked = jnp.where(s == mn, rowi, jnp.float32(K))
            amin = jnp.min(masked, axis=0, keepdims=True)
            oh = (rowi == amin).astype(jnp.float32)
            counts = counts + jnp.dot(
                oh, ones_b, preferred_element_type=jnp.float32,
                precision=jax.lax.Precision.HIGHEST)
            sums = sums + jnp.dot(oh, hb, preferred_element_type=jnp.float32,
                                  precision=jax.lax.Precision.HIGHEST)
            if with_sq:
                sqs = sqs + jnp.dot(oh, hb * hb,
                                    preferred_element_type=jnp.float32,
                                    precision=jax.lax.Precision.HIGHEST)
            return counts, sums, sqs

        z = jnp.zeros((K, H), jnp.float32)
        init = (jnp.zeros((K, 1), jnp.float32), z, z)
        counts, sums, sqs = jax.lax.fori_loop(0, _NBLK, blk, init)
        return counts.reshape(K), sums, sqs

    def iter_fn(t, c):
        counts, sums, _ = stats(c, False)
        newc = sums / jnp.maximum(counts, 1.0)[:, None]
        return jnp.where(counts[:, None] > 0, newc, c)

    c = jax.lax.fori_loop(0, KM_ITERS, iter_fn, c0)

    counts, _, sqs = stats(c, True)
    counts = jnp.maximum(counts, 1.0)
    sq = sqs / counts[:, None] - c * c
    vs = jnp.sqrt(jnp.maximum(sq, 1e-4))
    p = 0.5 / (vs * vs)

    const = jnp.sum(c * c * p, axis=1)
    cp2 = 2.0 * c * p
    wn = wn_ref[...]

    def fblk(i, _):
        rows = pl.ds(i * _BLK, _BLK)
        hb = h_ref[rows, :]
        hhb = hb * hb
        t1 = jax.lax.dot_general(p, hhb, (((1,), (1,)), ((), ())),
                                 preferred_element_type=jnp.float32,
                                 precision=jax.lax.Precision.HIGHEST)
        t2 = jax.lax.dot_general(cp2, hb, (((1,), (1,)), ((), ())),
                                 preferred_element_type=jnp.float32,
                                 precision=jax.lax.Precision.HIGHEST)
        logm = t2 - t1 - const[:, None]
        m = jnp.max(logm, axis=0, keepdims=True)
        e = jnp.exp(logm - m)
        frs = e / jnp.sum(e, axis=0, keepdims=True)
        fuzz = jax.lax.dot_general(frs, c, (((0,), (0,)), ((), ())),
                                   preferred_element_type=jnp.float32)
        out_ref[rows, :] = (jnp.dot(fuzz, wn,
                                    preferred_element_type=jnp.float32)
                            * dinvn_ref[rows, :])
        return 0

    jax.lax.fori_loop(0, _NBLK, fblk, 0)


def _layer_call(a0, a1, hs, dinv, b, wn, dinvn):
    return pl.pallas_call(
        _layer_body,
        out_shape=jax.ShapeDtypeStruct((N_NODES, H), jnp.float32),
        scratch_shapes=[pltpu.VMEM((N_NODES, H), jnp.float32)],
        interpret=_INTERPRET,
    )(a0, a1, hs, dinv, b, wn, dinvn)


from jax import lax
from jax.experimental.pallas import tpu_sc as plsc

_NC = 2
_NS = 16
_NW = _NC * _NS
_EPW = N_EDGES // _NW
_EC = 80
_NCH = _EPW // _EC
_CPT = 10
_RPT = N_NODES // _CPT
_ZR = 200
_ZRF = 2 * N_NODES // _CPT


def _sc_mesh():
    return plsc.VectorSubcoreMesh(core_axis_name="c", subcore_axis_name="s",
                                  num_cores=_NC, num_subcores=_NS)


def _deg_partials(dst, w):
    @functools.partial(
        pl.kernel,
        out_type=jax.ShapeDtypeStruct((_NC * 2 * N_NODES,), jnp.float32),
        mesh=_sc_mesh(),
        scratch_types=[
            pltpu.VMEM((_EPW,), jnp.int32),
            pltpu.VMEM((_EPW,), jnp.float32),
            pltpu.VMEM((_EC,), jnp.int32),
            pltpu.VMEM((_EC,), jnp.int32),
            pltpu.VMEM((_EC,), jnp.float32),
            pltpu.VMEM((_EC,), jnp.int32),
            pltpu.VMEM((_EC,), jnp.int32),
            pltpu.VMEM((_EC,), jnp.float32),
            pltpu.VMEM((_EC,), jnp.float32),
            pltpu.VMEM((_ZRF,), jnp.float32),
            pltpu.VMEM_SHARED((2 * N_NODES,), jnp.float32),
            pltpu.SemaphoreType.DMA,
            pltpu.SemaphoreType.DMA,
        ],
    )
    def deg(dst_hbm, w_hbm, out_hbm, dall, wall, de0, do0, wc0, de1, do1,
            wc1, one_v, z_v, shared, sem0, sem1):
        cid = lax.axis_index("c")
        sid = lax.axis_index("s")
        wid = sid * _NC + cid
        de = (de0, de1)
        do = (do0, do1)
        wc = (wc0, wc1)
        sems = (sem0, sem1)

        z16 = jnp.zeros((16,), jnp.float32)
        o16 = jnp.ones((16,), jnp.float32)

        def zi(i, _):
            z_v[pl.ds(i * 16, 16)] = z16
            return 0

        lax.fori_loop(0, _ZRF // 16, zi, 0)
        for g in range(_EC // 16):
            one_v[pl.ds(g * 16, 16)] = o16

        @pl.when(sid < _CPT)
        def _():
            zb = pl.multiple_of(sid * _ZRF, 8)
            pltpu.sync_copy(z_v, shared.at[pl.ds(zb, _ZRF)])

        base = pl.multiple_of(wid * _EPW, 8)
        pltpu.sync_copy(dst_hbm.at[pl.ds(base, _EPW)], dall)
        pltpu.sync_copy(w_hbm.at[pl.ds(base, _EPW)], wall)
        plsc.subcore_barrier()

        def issue(ch, p):
            for g in range(_EC // 16):
                sl = pl.ds(g * 16, 16)
                esl = pl.ds(ch * _EC + g * 16, 16)
                d2 = dall[esl] * 2
                de[p][sl] = d2
                do[p][sl] = d2 + 1
                wc[p][sl] = wall[esl]
            pltpu.async_copy(wc[p], shared.at[de[p]], sems[p], add=True)
            pltpu.async_copy(one_v, shared.at[do[p]], sems[p], add=True)

        def drain(p):
            pltpu.make_async_copy(wc[p], shared.at[de[p]], sems[p]).wait()
            pltpu.make_async_copy(one_v, shared.at[do[p]], sems[p]).wait()

        def do_deg(ch, par):
            @pl.when(ch >= 2)
            def _():
                drain(par)

            issue(ch, par)

        def pair(i, _):
            do_deg(2 * i, 0)
            do_deg(2 * i + 1, 1)
            return 0

        lax.fori_loop(0, _NCH // 2, pair, 0)
        do_deg(_NCH - 1, 0)
        drain(1)
        drain(0)
        plsc.subcore_barrier()

        @pl.when(sid < _CPT)
        def _():
            zb = pl.multiple_of(sid * _ZRF, 8)
            ob = pl.multiple_of(cid * 2 * N_NODES + sid * _ZRF, 8)
            pltpu.sync_copy(shared.at[pl.ds(zb, _ZRF)], z_v)
            pltpu.sync_copy(z_v, out_hbm.at[pl.ds(ob, _ZRF)])

    return deg(dst, w).reshape(_NC, N_NODES, 2)


def _agg(src, dst, w, rows, weighted):
    @functools.partial(
        pl.kernel,
        out_type=jax.ShapeDtypeStruct((_NC, N_NODES, H), jnp.float32),
        mesh=_sc_mesh(),
        scratch_types=[
            pltpu.VMEM((_EPW,), jnp.int32),
            pltpu.VMEM((_EPW,), jnp.int32),
            pltpu.VMEM((_EPW,), jnp.float32),
            pltpu.VMEM((_EC,), jnp.int32),
            pltpu.VMEM((_EC,), jnp.int32),
            pltpu.VMEM((_EC,), jnp.int32),
            pltpu.VMEM((_EC,), jnp.int32),
            pltpu.VMEM((_EC, H), jnp.float32),
            pltpu.VMEM((_EC, H), jnp.float32),
            pltpu.VMEM((_ZR, H), jnp.float32),
            pltpu.VMEM_SHARED((N_NODES, H), jnp.float32),
            pltpu.SemaphoreType.DMA,
            pltpu.SemaphoreType.DMA,
            pltpu.SemaphoreType.DMA,
            pltpu.SemaphoreType.DMA,
        ],
        compiler_params=pltpu.CompilerParams(use_tc_tiling_on_sc=False),
    )
    def agg(src_hbm, dst_hbm, w_hbm, rows_hbm, out_hbm,
            sall, dall, wall, si0, si1, di0, di1, rb0, rb1, z_v, shared,
            gs0, gs1, ss0, ss1):
        cid = lax.axis_index("c")
        sid = lax.axis_index("s")
        wid = sid * _NC + cid
        si = (si0, si1)
        di = (di0, di1)
        rb = (rb0, rb1)
        gs = (gs0, gs1)
        ss = (ss0, ss1)

        z16 = jnp.zeros((16,), jnp.float32)

        def zi(i, _):
            for q in range(H // 16):
                z_v[i, pl.ds(q * 16, 16)] = z16
            return 0

        lax.fori_loop(0, _ZR, zi, 0)

        @pl.when(sid < _CPT)
        def _():
            for j in range(_RPT // _ZR):
                r0 = pl.multiple_of(sid * _RPT + j * _ZR, 8)
                pltpu.sync_copy(z_v, shared.at[pl.ds(r0, _ZR)])

        base = pl.multiple_of(wid * _EPW, 8)
        pltpu.sync_copy(src_hbm.at[pl.ds(base, _EPW)], sall)
        pltpu.sync_copy(dst_hbm.at[pl.ds(base, _EPW)], dall)
        if weighted:
            pltpu.sync_copy(w_hbm.at[pl.ds(base, _EPW)], wall)
        plsc.subcore_barrier()

        def start_gather(ch, p):
            for g in range(_EC // 16):
                sl = pl.ds(g * 16, 16)
                esl = pl.ds(ch * _EC + g * 16, 16)
                si[p][sl] = sall[esl]
                di[p][sl] = dall[esl]
            pltpu.async_copy(rows_hbm.at[si[p]], rb[p], gs[p])

        start_gather(0, 0)

        def do_chunk(ch, par):
            q = 1 - par

            @pl.when(ch + 1 < _NCH)
            def _():
                @pl.when(ch >= 1)
                def _():
                    pltpu.make_async_copy(
                        rb[q], shared.at[di[q]], ss[q]).wait()

                start_gather(ch + 1, q)

            pltpu.make_async_copy(rows_hbm.at[si[par]], rb[par],
                                  gs[par]).wait()
            if weighted:
                for g in range(_EC // 16):
                    w16 = wall[pl.ds(ch * _EC + g * 16, 16)]
                    for e in range(16):
                        sp = w16.at[jnp.full((16,), e, jnp.int32)].get(
                            mode="promise_in_bounds")
                        r = g * 16 + e
                        for qq in range(H // 16):
                            sl = pl.ds(qq * 16, 16)
                            rb[par][r, sl] = rb[par][r, sl] * sp
            pltpu.async_copy(rb[par], shared.at[di[par]], ss[par], add=True)

        def pair(i, _):
            do_chunk(2 * i, 0)
            do_chunk(2 * i + 1, 1)
            return 0

        lax.fori_loop(0, _NCH // 2, pair, 0)
        do_chunk(_NCH - 1, 0)
        pltpu.make_async_copy(rb[1], shared.at[di[1]], ss[1]).wait()
        pltpu.make_async_copy(rb[0], shared.at[di[0]], ss[0]).wait()
        plsc.subcore_barrier()

        @pl.when(sid < _CPT)
        def _():
            for j in range(_RPT // _ZR):
                r0 = pl.multiple_of(sid * _RPT + j * _ZR, 8)
                pltpu.sync_copy(shared.at[pl.ds(r0, _ZR)], z_v)
                pltpu.sync_copy(z_v, out_hbm.at[cid, pl.ds(r0, _ZR)])

    a = agg(src, dst, w, rows)
    return a[0], a[1]


def kernel(x, edge_index, edge_weight, W1, b1, W2, b2):
    src = edge_index[0]
    dst = edge_index[1]
    degp = _deg_partials(dst, edge_weight)

    hs1, dinv = _prep_call(x, W1, degp)
    dinv1 = dinv[:, 0:1]
    dinv2 = dinv[:, 1:2]

    a0, a1 = _agg(src, dst, edge_weight, hs1, True)
    hs2 = _layer_call(a0, a1, hs1, dinv1, b1.reshape(1, H), W2, dinv2)

    b0p, b1p = _agg(src, dst, edge_weight, hs2, False)
    eye = jnp.eye(H, dtype=jnp.float32)
    ones = jnp.ones((N_NODES, 1), jnp.float32)
    out = _layer_call(b0p, b1p, hs2, dinv2, b2.reshape(1, H), eye, ones)
    return out


# speedup vs baseline: 28.6978x; 1.3845x over previous
import functools

import jax
import jax.numpy as jnp
from jax.experimental import pallas as pl
from jax.experimental.pallas import tpu as pltpu

N_NODES = 10000
N_EDGES = 320000
D_IN = 128
H = 64
K = 16
KM_ITERS = 10

_INIT_IDX = (0, 666, 1333, 1999, 2666, 3333, 3999, 4666,
             5332, 5999, 6666, 7332, 7999, 8665, 9332, 9999)

_INTERPRET = False


_BLK = 500
_NBLK = N_NODES // _BLK
_BLKS = 2000
_NBLKS = N_NODES // _BLKS


def _prep_body(x_ref, w1_ref, degs_ref, hs_ref, dinv_ref):
    w1 = w1_ref[...]

    def blk(i, _):
        rows = pl.ds(i * _BLK, _BLK)
        s = degs_ref[0, rows, :] + degs_ref[1, rows, :]
        dinv = jax.lax.rsqrt(1.0 + s)
        h = jnp.dot(x_ref[rows, :], w1, preferred_element_type=jnp.float32)
        hs_ref[rows, :] = h * dinv[:, 0:1]
        dinv_ref[rows, :] = dinv
        return 0

    jax.lax.fori_loop(0, _NBLK, blk, 0)


def _prep_call(x, w1, deg_partials):
    return pl.pallas_call(
        _prep_body,
        out_shape=(
            jax.ShapeDtypeStruct((N_NODES, H), jnp.float32),
            jax.ShapeDtypeStruct((N_NODES, 2), jnp.float32),
        ),
        interpret=_INTERPRET,
    )(x, w1, deg_partials)


def _layer_body(a0_ref, a1_ref, hs_ref, dinv_ref, b_ref, wn_ref, dinvn_ref,
                out_ref, h_ref):
    b = b_ref[...]

    def hblk(i, _):
        rows = pl.ds(i * _BLK, _BLK)
        h_ref[rows, :] = ((a0_ref[rows, :] + a1_ref[rows, :]
                           + hs_ref[rows, :]) * dinv_ref[rows, :] + b)
        return 0

    jax.lax.fori_loop(0, _NBLK, hblk, 0)

    c0 = jnp.concatenate([h_ref[i:i + 1, :] for i in _INIT_IDX], axis=0)

    ones_b = jnp.ones((_BLKS, 1), jnp.float32)

    def stats(c, with_sq):
        cn = jnp.sum(c * c, axis=1)

        def blk(i, carry):
            counts, sums, sqs = carry
            rows = pl.ds(i * _BLKS, _BLKS)
            hb = h_ref[rows, :]
            g = jax.lax.dot_general(c, hb, (((1,), (1,)), ((), ())),
                                    preferred_element_type=jnp.float32,
                                    precision=jax.lax.Precision.HIGHEST)
            s = cn[:, None] - 2.0 * g
            mn = jnp.min(s, axis=0, keepdims=True)
            rowi = jax.lax.broadcasted_iota(
                jnp.int32, (K, _BLKS), 0).astype(jnp.float32)
            masked = jnp.where(s == mn, rowi, jnp.float32(K))
            amin = jnp.min(masked, axis=0, keepdims=True)
            oh = (rowi == amin).astype(jnp.float32)
            counts = counts + jnp.dot(
                oh, ones_b, preferred_element_type=jnp.float32,
                precision=jax.lax.Precision.HIGHEST)
            sums = sums + jnp.dot(oh, hb, preferred_element_type=jnp.float32,
                                  precision=jax.lax.Precision.HIGHEST)
            if with_sq:
                sqs = sqs + jnp.dot(oh, hb * hb,
                                    preferred_element_type=jnp.float32,
                                    precision=jax.lax.Precision.HIGHEST)
            return counts, sums, sqs

        z = jnp.zeros((K, H), jnp.float32)
        init = (jnp.zeros((K, 1), jnp.float32), z, z)
        counts, sums, sqs = jax.lax.fori_loop(0, _NBLKS, blk, init)
        return counts.reshape(K), sums, sqs

    def iter_fn(t, c):
        counts, sums, _ = stats(c, False)
        newc = sums / jnp.maximum(counts, 1.0)[:, None]
        return jnp.where(counts[:, None] > 0, newc, c)

    c = jax.lax.fori_loop(0, KM_ITERS, iter_fn, c0)

    counts, _, sqs = stats(c, True)
    counts = jnp.maximum(counts, 1.0)
    sq = sqs / counts[:, None] - c * c
    vs = jnp.sqrt(jnp.maximum(sq, 1e-4))
    p = 0.5 / (vs * vs)

    const = jnp.sum(c * c * p, axis=1)
    cp2 = 2.0 * c * p
    wn = wn_ref[...]

    def fblk(i, _):
        rows = pl.ds(i * _BLK, _BLK)
        hb = h_ref[rows, :]
        hhb = hb * hb
        t1 = jax.lax.dot_general(p, hhb, (((1,), (1,)), ((), ())),
                                 preferred_element_type=jnp.float32,
                                 precision=jax.lax.Precision.HIGHEST)
        t2 = jax.lax.dot_general(cp2, hb, (((1,), (1,)), ((), ())),
                                 preferred_element_type=jnp.float32,
                                 precision=jax.lax.Precision.HIGHEST)
        logm = t2 - t1 - const[:, None]
        m = jnp.max(logm, axis=0, keepdims=True)
        e = jnp.exp(logm - m)
        frs = e / jnp.sum(e, axis=0, keepdims=True)
        fuzz = jax.lax.dot_general(frs, c, (((0,), (0,)), ((), ())),
                                   preferred_element_type=jnp.float32)
        out_ref[rows, :] = (jnp.dot(fuzz, wn,
                                    preferred_element_type=jnp.float32)
                            * dinvn_ref[rows, :])
        return 0

    jax.lax.fori_loop(0, _NBLK, fblk, 0)


def _layer_call(a0, a1, hs, dinv, b, wn, dinvn):
    return pl.pallas_call(
        _layer_body,
        out_shape=jax.ShapeDtypeStruct((N_NODES, H), jnp.float32),
        scratch_shapes=[pltpu.VMEM((N_NODES, H), jnp.float32)],
        interpret=_INTERPRET,
    )(a0, a1, hs, dinv, b, wn, dinvn)


from jax import lax
from jax.experimental.pallas import tpu_sc as plsc

_NC = 2
_NS = 16
_NW = _NC * _NS
_EPW = N_EDGES // _NW
_EC = 80
_NCH = _EPW // _EC
_CPT = 10
_RPT = N_NODES // _CPT
_ZR = 200
_ZRF = 2 * N_NODES // _CPT


def _sc_mesh():
    return plsc.VectorSubcoreMesh(core_axis_name="c", subcore_axis_name="s",
                                  num_cores=_NC, num_subcores=_NS)


def _deg_partials(dst, w):
    @functools.partial(
        pl.kernel,
        out_type=jax.ShapeDtypeStruct((_NC * 2 * N_NODES,), jnp.float32),
        mesh=_sc_mesh(),
        scratch_types=[
            pltpu.VMEM((_EPW,), jnp.int32),
            pltpu.VMEM((_EPW,), jnp.float32),
            pltpu.VMEM((_EC,), jnp.int32),
            pltpu.VMEM((_EC,), jnp.int32),
            pltpu.VMEM((_EC,), jnp.float32),
            pltpu.VMEM((_EC,), jnp.int32),
            pltpu.VMEM((_EC,), jnp.int32),
            pltpu.VMEM((_EC,), jnp.float32),
            pltpu.VMEM((_EC,), jnp.float32),
            pltpu.VMEM((_ZRF,), jnp.float32),
            pltpu.VMEM_SHARED((2 * N_NODES,), jnp.float32),
            pltpu.SemaphoreType.DMA,
            pltpu.SemaphoreType.DMA,
        ],
    )
    def deg(dst_hbm, w_hbm, out_hbm, dall, wall, de0, do0, wc0, de1, do1,
            wc1, one_v, z_v, shared, sem0, sem1):
        cid = lax.axis_index("c")
        sid = lax.axis_index("s")
        wid = sid * _NC + cid
        de = (de0, de1)
        do = (do0, do1)
        wc = (wc0, wc1)
        sems = (sem0, sem1)

        z16 = jnp.zeros((16,), jnp.float32)
        o16 = jnp.ones((16,), jnp.float32)

        def zi(i, _):
            z_v[pl.ds(i * 16, 16)] = z16
            return 0

        lax.fori_loop(0, _ZRF // 16, zi, 0)
        for g in range(_EC // 16):
            one_v[pl.ds(g * 16, 16)] = o16

        @pl.when(sid < _CPT)
        def _():
            zb = pl.multiple_of(sid * _ZRF, 8)
            pltpu.sync_copy(z_v, shared.at[pl.ds(zb, _ZRF)])

        base = pl.multiple_of(wid * _EPW, 8)
        pltpu.sync_copy(dst_hbm.at[pl.ds(base, _EPW)], dall)
        pltpu.sync_copy(w_hbm.at[pl.ds(base, _EPW)], wall)
        plsc.subcore_barrier()

        def issue(ch, p):
            for g in range(_EC // 16):
                sl = pl.ds(g * 16, 16)
                esl = pl.ds(ch * _EC + g * 16, 16)
                d2 = dall[esl] * 2
                de[p][sl] = d2
                do[p][sl] = d2 + 1
                wc[p][sl] = wall[esl]
            pltpu.async_copy(wc[p], shared.at[de[p]], sems[p], add=True)
            pltpu.async_copy(one_v, shared.at[do[p]], sems[p], add=True)

        def drain(p):
            pltpu.make_async_copy(wc[p], shared.at[de[p]], sems[p]).wait()
            pltpu.make_async_copy(one_v, shared.at[do[p]], sems[p]).wait()

        def do_deg(ch, par):
            @pl.when(ch >= 2)
            def _():
                drain(par)

            issue(ch, par)

        def pair(i, _):
            do_deg(2 * i, 0)
            do_deg(2 * i + 1, 1)
            return 0

        lax.fori_loop(0, _NCH // 2, pair, 0)
        do_deg(_NCH - 1, 0)
        drain(1)
        drain(0)
        plsc.subcore_barrier()

        @pl.when(sid < _CPT)
        def _():
            zb = pl.multiple_of(sid * _ZRF, 8)
            ob = pl.multiple_of(cid * 2 * N_NODES + sid * _ZRF, 8)
            pltpu.sync_copy(shared.at[pl.ds(zb, _ZRF)], z_v)
            pltpu.sync_copy(z_v, out_hbm.at[pl.ds(ob, _ZRF)])

    return deg(dst, w).reshape(_NC, N_NODES, 2)


def _agg(src, dst, w, rows, weighted):
    @functools.partial(
        pl.kernel,
        out_type=jax.ShapeDtypeStruct((_NC, N_NODES, H), jnp.float32),
        mesh=_sc_mesh(),
        scratch_types=[
            pltpu.VMEM((_EPW,), jnp.int32),
            pltpu.VMEM((_EPW,), jnp.int32),
            pltpu.VMEM((_EPW,), jnp.float32),
            pltpu.VMEM((_EC,), jnp.int32),
            pltpu.VMEM((_EC,), jnp.int32),
            pltpu.VMEM((_EC,), jnp.int32),
            pltpu.VMEM((_EC,), jnp.int32),
            pltpu.VMEM((_EC, H), jnp.float32),
            pltpu.VMEM((_EC, H), jnp.float32),
            pltpu.VMEM((_ZR, H), jnp.float32),
            pltpu.VMEM_SHARED((N_NODES, H), jnp.float32),
            pltpu.SemaphoreType.DMA,
            pltpu.SemaphoreType.DMA,
            pltpu.SemaphoreType.DMA,
            pltpu.SemaphoreType.DMA,
        ],
        compiler_params=pltpu.CompilerParams(use_tc_tiling_on_sc=False),
    )
    def agg(src_hbm, dst_hbm, w_hbm, rows_hbm, out_hbm,
            sall, dall, wall, si0, si1, di0, di1, rb0, rb1, z_v, shared,
            gs0, gs1, ss0, ss1):
        cid = lax.axis_index("c")
        sid = lax.axis_index("s")
        wid = sid * _NC + cid
        si = (si0, si1)
        di = (di0, di1)
        rb = (rb0, rb1)
        gs = (gs0, gs1)
        ss = (ss0, ss1)

        z16 = jnp.zeros((16,), jnp.float32)

        def zi(i, _):
            for q in range(H // 16):
                z_v[i, pl.ds(q * 16, 16)] = z16
            return 0

        lax.fori_loop(0, _ZR, zi, 0)

        @pl.when(sid < _CPT)
        def _():
            for j in range(_RPT // _ZR):
                r0 = pl.multiple_of(sid * _RPT + j * _ZR, 8)
                pltpu.sync_copy(z_v, shared.at[pl.ds(r0, _ZR)])

        base = pl.multiple_of(wid * _EPW, 8)
        pltpu.sync_copy(src_hbm.at[pl.ds(base, _EPW)], sall)
        pltpu.sync_copy(dst_hbm.at[pl.ds(base, _EPW)], dall)
        if weighted:
            pltpu.sync_copy(w_hbm.at[pl.ds(base, _EPW)], wall)
        plsc.subcore_barrier()

        def start_gather(ch, p):
            for g in range(_EC // 16):
                sl = pl.ds(g * 16, 16)
                esl = pl.ds(ch * _EC + g * 16, 16)
                si[p][sl] = sall[esl]
                di[p][sl] = dall[esl]
            pltpu.async_copy(rows_hbm.at[si[p]], rb[p], gs[p])

        start_gather(0, 0)

        def do_chunk(ch, par):
            q = 1 - par

            @pl.when(ch + 1 < _NCH)
            def _():
                @pl.when(ch >= 1)
                def _():
                    pltpu.make_async_copy(
                        rb[q], shared.at[di[q]], ss[q]).wait()

                start_gather(ch + 1, q)

            pltpu.make_async_copy(rows_hbm.at[si[par]], rb[par],
                                  gs[par]).wait()
            if weighted:
                for g in range(_EC // 16):
                    w16 = wall[pl.ds(ch * _EC + g * 16, 16)]
                    for e in range(16):
                        sp = w16.at[jnp.full((16,), e, jnp.int32)].get(
                            mode="promise_in_bounds")
                        r = g * 16 + e
                        for qq in range(H // 16):
                            sl = pl.ds(qq * 16, 16)
                            rb[par][r, sl] = rb[par][r, sl] * sp
            pltpu.async_copy(rb[par], shared.at[di[par]], ss[par], add=True)

        def pair(i, _):
            do_chunk(2 * i, 0)
            do_chunk(2 * i + 1, 1)
            return 0

        lax.fori_loop(0, _NCH // 2, pair, 0)
        do_chunk(_NCH - 1, 0)
        pltpu.make_async_copy(rb[1], shared.at[di[1]], ss[1]).wait()
        pltpu.make_async_copy(rb[0], shared.at[di[0]], ss[0]).wait()
        plsc.subcore_barrier()

        @pl.when(sid < _CPT)
        def _():
            for j in range(_RPT // _ZR):
                r0 = pl.multiple_of(sid * _RPT + j * _ZR, 8)
                pltpu.sync_copy(shared.at[pl.ds(r0, _ZR)], z_v)
                pltpu.sync_copy(z_v, out_hbm.at[cid, pl.ds(r0, _ZR)])

    a = agg(src, dst, w, rows)
    return a[0], a[1]


def kernel(x, edge_index, edge_weight, W1, b1, W2, b2):
    src = edge_index[0]
    dst = edge_index[1]
    degp = _deg_partials(dst, edge_weight)

    hs1, dinv = _prep_call(x, W1, degp)
    dinv1 = dinv[:, 0:1]
    dinv2 = dinv[:, 1:2]

    a0, a1 = _agg(src, dst, edge_weight, hs1, True)
    hs2 = _layer_call(a0, a1, hs1, dinv1, b1.reshape(1, H), W2, dinv2)

    b0p, b1p = _agg(src, dst, edge_weight, hs2, False)
    eye = jnp.eye(H, dtype=jnp.float32)
    ones = jnp.ones((N_NODES, 1), jnp.float32)
    out = _layer_call(b0p, b1p, hs2, dinv2, b2.reshape(1, H), eye, ones)
    return out


# speedup vs baseline: 30.3517x; 1.0576x over previous
import functools

import jax
import jax.numpy as jnp
from jax.experimental import pallas as pl
from jax.experimental.pallas import tpu as pltpu

N_NODES = 10000
N_EDGES = 320000
D_IN = 128
H = 64
K = 16
KM_ITERS = 10

_INIT_IDX = (0, 666, 1333, 1999, 2666, 3333, 3999, 4666,
             5332, 5999, 6666, 7332, 7999, 8665, 9332, 9999)

_INTERPRET = False


_BLK = 500
_NBLK = N_NODES // _BLK
_BLKS = 2500
_NBLKS = N_NODES // _BLKS
_BLKF = 2000
_NBLKF = N_NODES // _BLKF


def _prep_body(x_ref, w1_ref, degs_ref, hs_ref, dinv_ref):
    w1 = w1_ref[...]

    def blk(i, _):
        rows = pl.ds(i * _BLK, _BLK)
        s = degs_ref[0, rows, :] + degs_ref[1, rows, :]
        dinv = jax.lax.rsqrt(1.0 + s)
        h = jnp.dot(x_ref[rows, :], w1, preferred_element_type=jnp.float32)
        hs_ref[rows, :] = h * dinv[:, 0:1]
        dinv_ref[rows, :] = dinv
        return 0

    jax.lax.fori_loop(0, _NBLK, blk, 0)


def _prep_call(x, w1, deg_partials):
    return pl.pallas_call(
        _prep_body,
        out_shape=(
            jax.ShapeDtypeStruct((N_NODES, H), jnp.float32),
            jax.ShapeDtypeStruct((N_NODES, 2), jnp.float32),
        ),
        interpret=_INTERPRET,
    )(x, w1, deg_partials)


def _layer_body(a0_ref, a1_ref, hs_ref, dinv_ref, b_ref, wn_ref, dinvn_ref,
                out_ref, h_ref):
    b = b_ref[...]

    def hblk(i, _):
        rows = pl.ds(i * _BLKF, _BLKF)
        h_ref[rows, :] = ((a0_ref[rows, :] + a1_ref[rows, :]
                           + hs_ref[rows, :]) * dinv_ref[rows, :] + b)
        return 0

    jax.lax.fori_loop(0, _NBLKF, hblk, 0)

    c0 = jnp.concatenate([h_ref[i:i + 1, :] for i in _INIT_IDX], axis=0)

    ones_b = jnp.ones((_BLKS, 1), jnp.float32)

    def stats(c, with_sq):
        cn = jnp.sum(c * c, axis=1)

        def blk(i, carry):
            counts, sums, sqs = carry
            rows = pl.ds(i * _BLKS, _BLKS)
            hb = h_ref[rows, :]
            g = jax.lax.dot_general(c, hb, (((1,), (1,)), ((), ())),
                                    preferred_element_type=jnp.float32,
                                    precision=jax.lax.Precision.HIGHEST)
            s = cn[:, None] - 2.0 * g
            mn = jnp.min(s, axis=0, keepdims=True)
            rowi = jax.lax.broadcasted_iota(
                jnp.int32, (K, _BLKS), 0).astype(jnp.float32)
            masked = jnp.where(s == mn, rowi, jnp.float32(K))
            amin = jnp.min(masked, axis=0, keepdims=True)
            oh = (rowi == amin).astype(jnp.float32)
            counts = counts + jnp.dot(
                oh, ones_b, preferred_element_type=jnp.float32,
                precision=jax.lax.Precision.HIGHEST)
            sums = sums + jnp.dot(oh, hb, preferred_element_type=jnp.float32,
                                  precision=jax.lax.Precision.HIGHEST)
            if with_sq:
                sqs = sqs + jnp.dot(oh, hb * hb,
                                    preferred_element_type=jnp.float32,
                                    precision=jax.lax.Precision.HIGHEST)
            return counts, sums, sqs

        z = jnp.zeros((K, H), jnp.float32)
        init = (jnp.zeros((K, 1), jnp.float32), z, z)
        counts, sums, sqs = jax.lax.fori_loop(0, _NBLKS, blk, init)
        return counts.reshape(K), sums, sqs

    def iter_fn(t, c):
        counts, sums, _ = stats(c, False)
        newc = sums / jnp.maximum(counts, 1.0)[:, None]
        return jnp.where(counts[:, None] > 0, newc, c)

    c = jax.lax.fori_loop(0, KM_ITERS, iter_fn, c0)

    counts, _, sqs = stats(c, True)
    counts = jnp.maximum(counts, 1.0)
    sq = sqs / counts[:, None] - c * c
    vs = jnp.sqrt(jnp.maximum(sq, 1e-4))
    p = 0.5 / (vs * vs)

    const = jnp.sum(c * c * p, axis=1)
    cp2 = 2.0 * c * p
    wn = wn_ref[...]

    def fblk(i, _):
        rows = pl.ds(i * _BLKF, _BLKF)
        hb = h_ref[rows, :]
        hhb = hb * hb
        t1 = jax.lax.dot_general(p, hhb, (((1,), (1,)), ((), ())),
                                 preferred_element_type=jnp.float32,
                                 precision=jax.lax.Precision.HIGHEST)
        t2 = jax.lax.dot_general(cp2, hb, (((1,), (1,)), ((), ())),
                                 preferred_element_type=jnp.float32,
                                 precision=jax.lax.Precision.HIGHEST)
        logm = t2 - t1 - const[:, None]
        m = jnp.max(logm, axis=0, keepdims=True)
        e = jnp.exp(logm - m)
        frs = e / jnp.sum(e, axis=0, keepdims=True)
        fuzz = jax.lax.dot_general(frs, c, (((0,), (0,)), ((), ())),
                                   preferred_element_type=jnp.float32)
        out_ref[rows, :] = (jnp.dot(fuzz, wn,
                                    preferred_element_type=jnp.float32)
                            * dinvn_ref[rows, :])
        return 0

    jax.lax.fori_loop(0, _NBLKF, fblk, 0)


def _layer_call(a0, a1, hs, dinv, b, wn, dinvn):
    return pl.pallas_call(
        _layer_body,
        out_shape=jax.ShapeDtypeStruct((N_NODES, H), jnp.float32),
        scratch_shapes=[pltpu.VMEM((N_NODES, H), jnp.float32)],
        interpret=_INTERPRET,
    )(a0, a1, hs, dinv, b, wn, dinvn)


from jax import lax
from jax.experimental.pallas import tpu_sc as plsc

_NC = 2
_NS = 16
_NW = _NC * _NS
_EPW = N_EDGES // _NW
_EC = 80
_NCH = _EPW // _EC
_CPT = 10
_RPT = N_NODES // _CPT
_ZR = 200
_ZRF = 2 * N_NODES // _CPT


def _sc_mesh():
    return plsc.VectorSubcoreMesh(core_axis_name="c", subcore_axis_name="s",
                                  num_cores=_NC, num_subcores=_NS)


def _deg_partials(dst, w):
    @functools.partial(
        pl.kernel,
        out_type=jax.ShapeDtypeStruct((_NC * 2 * N_NODES,), jnp.float32),
        mesh=_sc_mesh(),
        scratch_types=[
            pltpu.VMEM((_EPW,), jnp.int32),
            pltpu.VMEM((_EPW,), jnp.float32),
            pltpu.VMEM((_EC,), jnp.int32),
            pltpu.VMEM((_EC,), jnp.int32),
            pltpu.VMEM((_EC,), jnp.float32),
            pltpu.VMEM((_EC,), jnp.int32),
            pltpu.VMEM((_EC,), jnp.int32),
            pltpu.VMEM((_EC,), jnp.float32),
            pltpu.VMEM((_EC,), jnp.float32),
            pltpu.VMEM((_ZRF,), jnp.float32),
            pltpu.VMEM_SHARED((2 * N_NODES,), jnp.float32),
            pltpu.SemaphoreType.DMA,
            pltpu.SemaphoreType.DMA,
        ],
    )
    def deg(dst_hbm, w_hbm, out_hbm, dall, wall, de0, do0, wc0, de1, do1,
            wc1, one_v, z_v, shared, sem0, sem1):
        cid = lax.axis_index("c")
        sid = lax.axis_index("s")
        wid = sid * _NC + cid
        de = (de0, de1)
        do = (do0, do1)
        wc = (wc0, wc1)
        sems = (sem0, sem1)

        z16 = jnp.zeros((16,), jnp.float32)
        o16 = jnp.ones((16,), jnp.float32)

        def zi(i, _):
            z_v[pl.ds(i * 16, 16)] = z16
            return 0

        lax.fori_loop(0, _ZRF // 16, zi, 0)
        for g in range(_EC // 16):
            one_v[pl.ds(g * 16, 16)] = o16

        @pl.when(sid < _CPT)
        def _():
            zb = pl.multiple_of(sid * _ZRF, 8)
            pltpu.sync_copy(z_v, shared.at[pl.ds(zb, _ZRF)])

        base = pl.multiple_of(wid * _EPW, 8)
        pltpu.sync_copy(dst_hbm.at[pl.ds(base, _EPW)], dall)
        pltpu.sync_copy(w_hbm.at[pl.ds(base, _EPW)], wall)
        plsc.subcore_barrier()

        def issue(ch, p):
            for g in range(_EC // 16):
                sl = pl.ds(g * 16, 16)
                esl = pl.ds(ch * _EC + g * 16, 16)
                d2 = dall[esl] * 2
                de[p][sl] = d2
                do[p][sl] = d2 + 1
                wc[p][sl] = wall[esl]
            pltpu.async_copy(wc[p], shared.at[de[p]], sems[p], add=True)
            pltpu.async_copy(one_v, shared.at[do[p]], sems[p], add=True)

        def drain(p):
            pltpu.make_async_copy(wc[p], shared.at[de[p]], sems[p]).wait()
            pltpu.make_async_copy(one_v, shared.at[do[p]], sems[p]).wait()

        def do_deg(ch, par):
            @pl.when(ch >= 2)
            def _():
                drain(par)

            issue(ch, par)

        def pair(i, _):
            do_deg(2 * i, 0)
            do_deg(2 * i + 1, 1)
            return 0

        lax.fori_loop(0, _NCH // 2, pair, 0)
        do_deg(_NCH - 1, 0)
        drain(1)
        drain(0)
        plsc.subcore_barrier()

        @pl.when(sid < _CPT)
        def _():
            zb = pl.multiple_of(sid * _ZRF, 8)
            ob = pl.multiple_of(cid * 2 * N_NODES + sid * _ZRF, 8)
            pltpu.sync_copy(shared.at[pl.ds(zb, _ZRF)], z_v)
            pltpu.sync_copy(z_v, out_hbm.at[pl.ds(ob, _ZRF)])

    return deg(dst, w).reshape(_NC, N_NODES, 2)


def _agg(src, dst, w, rows, weighted):
    @functools.partial(
        pl.kernel,
        out_type=jax.ShapeDtypeStruct((_NC, N_NODES, H), jnp.float32),
        mesh=_sc_mesh(),
        scratch_types=[
            pltpu.VMEM((_EPW,), jnp.int32),
            pltpu.VMEM((_EPW,), jnp.int32),
            pltpu.VMEM((_EPW,), jnp.float32),
            pltpu.VMEM((_EC,), jnp.int32),
            pltpu.VMEM((_EC,), jnp.int32),
            pltpu.VMEM((_EC,), jnp.int32),
            pltpu.VMEM((_EC,), jnp.int32),
            pltpu.VMEM((_EC, H), jnp.float32),
            pltpu.VMEM((_EC, H), jnp.float32),
            pltpu.VMEM((_ZR, H), jnp.float32),
            pltpu.VMEM_SHARED((N_NODES, H), jnp.float32),
            pltpu.SemaphoreType.DMA,
            pltpu.SemaphoreType.DMA,
            pltpu.SemaphoreType.DMA,
            pltpu.SemaphoreType.DMA,
        ],
        compiler_params=pltpu.CompilerParams(use_tc_tiling_on_sc=False),
    )
    def agg(src_hbm, dst_hbm, w_hbm, rows_hbm, out_hbm,
            sall, dall, wall, si0, si1, di0, di1, rb0, rb1, z_v, shared,
            gs0, gs1, ss0, ss1):
        cid = lax.axis_index("c")
        sid = lax.axis_index("s")
        wid = sid * _NC + cid
        si = (si0, si1)
        di = (di0, di1)
        rb = (rb0, rb1)
        gs = (gs0, gs1)
        ss = (ss0, ss1)

        z16 = jnp.zeros((16,), jnp.float32)

        def zi(i, _):
            for q in range(H // 16):
                z_v[i, pl.ds(q * 16, 16)] = z16
            return 0

        lax.fori_loop(0, _ZR, zi, 0)

        @pl.when(sid < _CPT)
        def _():
            for j in range(_RPT // _ZR):
                r0 = pl.multiple_of(sid * _RPT + j * _ZR, 8)
                pltpu.sync_copy(z_v, shared.at[pl.ds(r0, _ZR)])

        base = pl.multiple_of(wid * _EPW, 8)
        pltpu.sync_copy(src_hbm.at[pl.ds(base, _EPW)], sall)
        pltpu.sync_copy(dst_hbm.at[pl.ds(base, _EPW)], dall)
        if weighted:
            pltpu.sync_copy(w_hbm.at[pl.ds(base, _EPW)], wall)
        plsc.subcore_barrier()

        def start_gather(ch, p):
            for g in range(_EC // 16):
                sl = pl.ds(g * 16, 16)
                esl = pl.ds(ch * _EC + g * 16, 16)
                si[p][sl] = sall[esl]
                di[p][sl] = dall[esl]
            pltpu.async_copy(rows_hbm.at[si[p]], rb[p], gs[p])

        start_gather(0, 0)

        def do_chunk(ch, par):
            q = 1 - par

            @pl.when(ch + 1 < _NCH)
            def _():
                @pl.when(ch >= 1)
                def _():
                    pltpu.make_async_copy(
                        rb[q], shared.at[di[q]], ss[q]).wait()

                start_gather(ch + 1, q)

            pltpu.make_async_copy(rows_hbm.at[si[par]], rb[par],
                                  gs[par]).wait()
            if weighted:
                for g in range(_EC // 16):
                    w16 = wall[pl.ds(ch * _EC + g * 16, 16)]
                    for e in range(16):
                        sp = w16.at[jnp.full((16,), e, jnp.int32)].get(
                            mode="promise_in_bounds")
                        r = g * 16 + e
                        for qq in range(H // 16):
                            sl = pl.ds(qq * 16, 16)
                            rb[par][r, sl] = rb[par][r, sl] * sp
            pltpu.async_copy(rb[par], shared.at[di[par]], ss[par], add=True)

        def pair(i, _):
            do_chunk(2 * i, 0)
            do_chunk(2 * i + 1, 1)
            return 0

        lax.fori_loop(0, _NCH // 2, pair, 0)
        do_chunk(_NCH - 1, 0)
        pltpu.make_async_copy(rb[1], shared.at[di[1]], ss[1]).wait()
        pltpu.make_async_copy(rb[0], shared.at[di[0]], ss[0]).wait()
        plsc.subcore_barrier()

        @pl.when(sid < _CPT)
        def _():
            for j in range(_RPT // _ZR):
                r0 = pl.multiple_of(sid * _RPT + j * _ZR, 8)
                pltpu.sync_copy(shared.at[pl.ds(r0, _ZR)], z_v)
                pltpu.sync_copy(z_v, out_hbm.at[cid, pl.ds(r0, _ZR)])

    a = agg(src, dst, w, rows)
    return a[0], a[1]


def kernel(x, edge_index, edge_weight, W1, b1, W2, b2):
    src = edge_index[0]
    dst = edge_index[1]
    degp = _deg_partials(dst, edge_weight)

    hs1, dinv = _prep_call(x, W1, degp)
    dinv1 = dinv[:, 0:1]
    dinv2 = dinv[:, 1:2]

    a0, a1 = _agg(src, dst, edge_weight, hs1, True)
    hs2 = _layer_call(a0, a1, hs1, dinv1, b1.reshape(1, H), W2, dinv2)

    b0p, b1p = _agg(src, dst, edge_weight, hs2, False)
    eye = jnp.eye(H, dtype=jnp.float32)
    ones = jnp.ones((N_NODES, 1), jnp.float32)
    out = _layer_call(b0p, b1p, hs2, dinv2, b2.reshape(1, H), eye, ones)
    return out


# speedup vs baseline: 30.6212x; 1.0089x over previous
import functools

import jax
import jax.numpy as jnp
from jax.experimental import pallas as pl
from jax.experimental.pallas import tpu as pltpu

N_NODES = 10000
N_EDGES = 320000
D_IN = 128
H = 64
K = 16
KM_ITERS = 10

_INIT_IDX = (0, 666, 1333, 1999, 2666, 3333, 3999, 4666,
             5332, 5999, 6666, 7332, 7999, 8665, 9332, 9999)

_INTERPRET = False


_BLK = 500
_NBLK = N_NODES // _BLK
_BLKS = 2500
_NBLKS = N_NODES // _BLKS
_BLKF = 2000
_NBLKF = N_NODES // _BLKF


def _prep_body(x_ref, w1_ref, degs_ref, hs_ref, dinv_ref):
    w1 = w1_ref[...]

    def blk(i, _):
        rows = pl.ds(i * _BLK, _BLK)
        s = degs_ref[0, rows, :] + degs_ref[1, rows, :]
        dinv = jax.lax.rsqrt(1.0 + s)
        h = jnp.dot(x_ref[rows, :], w1, preferred_element_type=jnp.float32)
        hs_ref[rows, :] = h * dinv[:, 0:1]
        dinv_ref[rows, :] = dinv
        return 0

    jax.lax.fori_loop(0, _NBLK, blk, 0)


def _prep_call(x, w1, deg_partials):
    return pl.pallas_call(
        _prep_body,
        out_shape=(
            jax.ShapeDtypeStruct((N_NODES, H), jnp.float32),
            jax.ShapeDtypeStruct((N_NODES, 2), jnp.float32),
        ),
        interpret=_INTERPRET,
    )(x, w1, deg_partials)


def _layer_body(final, *refs):
    if final:
        a0_ref, a1_ref, hs_ref, dinv_ref, b_ref, out_ref, h_ref = refs
        wn_ref = dinvn_ref = None
    else:
        (a0_ref, a1_ref, hs_ref, dinv_ref, b_ref, wn_ref, dinvn_ref,
         out_ref, h_ref) = refs
    b = b_ref[...]

    def hblk(i, _):
        rows = pl.ds(i * _BLKF, _BLKF)
        h_ref[rows, :] = ((a0_ref[rows, :] + a1_ref[rows, :]
                           + hs_ref[rows, :]) * dinv_ref[rows, :] + b)
        return 0

    jax.lax.fori_loop(0, _NBLKF, hblk, 0)

    c0 = jnp.concatenate([h_ref[i:i + 1, :] for i in _INIT_IDX], axis=0)

    ones_b = jnp.ones((_BLKS, 1), jnp.float32)

    def stats(c, with_sq):
        cn = jnp.sum(c * c, axis=1)

        def blk(i, carry):
            counts, sums, sqs = carry
            rows = pl.ds(i * _BLKS, _BLKS)
            hb = h_ref[rows, :]
            g = jax.lax.dot_general(c, hb, (((1,), (1,)), ((), ())),
                                    preferred_element_type=jnp.float32,
                                    precision=jax.lax.Precision.HIGHEST)
            s = cn[:, None] - 2.0 * g
            mn = jnp.min(s, axis=0, keepdims=True)
            rowi = jax.lax.broadcasted_iota(
                jnp.int32, (K, _BLKS), 0).astype(jnp.float32)
            masked = jnp.where(s == mn, rowi, jnp.float32(K))
            amin = jnp.min(masked, axis=0, keepdims=True)
            oh = (rowi == amin).astype(jnp.float32)
            counts = counts + jnp.dot(
                oh, ones_b, preferred_element_type=jnp.float32,
                precision=jax.lax.Precision.HIGHEST)
            sums = sums + jnp.dot(oh, hb, preferred_element_type=jnp.float32,
                                  precision=jax.lax.Precision.HIGHEST)
            if with_sq:
                sqs = sqs + jnp.dot(oh, hb * hb,
                                    preferred_element_type=jnp.float32,
                                    precision=jax.lax.Precision.HIGHEST)
            return counts, sums, sqs

        z = jnp.zeros((K, H), jnp.float32)
        init = (jnp.zeros((K, 1), jnp.float32), z, z)
        counts, sums, sqs = jax.lax.fori_loop(0, _NBLKS, blk, init)
        return counts.reshape(K), sums, sqs

    def iter_fn(t, c):
        counts, sums, _ = stats(c, False)
        newc = sums / jnp.maximum(counts, 1.0)[:, None]
        return jnp.where(counts[:, None] > 0, newc, c)

    c = jax.lax.fori_loop(0, KM_ITERS, iter_fn, c0)

    counts, _, sqs = stats(c, True)
    counts = jnp.maximum(counts, 1.0)
    sq = sqs / counts[:, None] - c * c
    vs = jnp.sqrt(jnp.maximum(sq, 1e-4))
    p = 0.5 / (vs * vs)

    const = jnp.sum(c * c * p, axis=1)
    cp2 = 2.0 * c * p
    wn = None if final else wn_ref[...]

    def fblk(i, _):
        rows = pl.ds(i * _BLKF, _BLKF)
        hb = h_ref[rows, :]
        hhb = hb * hb
        t1 = jax.lax.dot_general(p, hhb, (((1,), (1,)), ((), ())),
                                 preferred_element_type=jnp.float32,
                                 precision=jax.lax.Precision.HIGHEST)
        t2 = jax.lax.dot_general(cp2, hb, (((1,), (1,)), ((), ())),
                                 preferred_element_type=jnp.float32,
                                 precision=jax.lax.Precision.HIGHEST)
        logm = t2 - t1 - const[:, None]
        m = jnp.max(logm, axis=0, keepdims=True)
        e = jnp.exp(logm - m)
        frs = e / jnp.sum(e, axis=0, keepdims=True)
        fuzz = jax.lax.dot_general(frs, c, (((0,), (0,)), ((), ())),
                                   preferred_element_type=jnp.float32)
        if final:
            out_ref[rows, :] = fuzz
        else:
            out_ref[rows, :] = (jnp.dot(fuzz, wn,
                                        preferred_element_type=jnp.float32)
                                * dinvn_ref[rows, :])
        return 0

    jax.lax.fori_loop(0, _NBLKF, fblk, 0)


def _layer_call(a0, a1, hs, dinv, b, wn=None, dinvn=None):
    final = wn is None
    args = (a0, a1, hs, dinv, b) if final else (a0, a1, hs, dinv, b, wn,
                                                dinvn)
    return pl.pallas_call(
        functools.partial(_layer_body, final),
        out_shape=jax.ShapeDtypeStruct((N_NODES, H), jnp.float32),
        scratch_shapes=[pltpu.VMEM((N_NODES, H), jnp.float32)],
        interpret=_INTERPRET,
    )(*args)


from jax import lax
from jax.experimental.pallas import tpu_sc as plsc

_NC = 2
_NS = 16
_NW = _NC * _NS
_EPW = N_EDGES // _NW
_EC = 80
_NCH = _EPW // _EC
_CPT = 10
_RPT = N_NODES // _CPT
_ZR = 200
_ZRF = 2 * N_NODES // _CPT


def _sc_mesh():
    return plsc.VectorSubcoreMesh(core_axis_name="c", subcore_axis_name="s",
                                  num_cores=_NC, num_subcores=_NS)


def _deg_partials(dst, w):
    @functools.partial(
        pl.kernel,
        out_type=jax.ShapeDtypeStruct((_NC * 2 * N_NODES,), jnp.float32),
        mesh=_sc_mesh(),
        scratch_types=[
            pltpu.VMEM((_EPW,), jnp.int32),
            pltpu.VMEM((_EPW,), jnp.float32),
            pltpu.VMEM((_EC,), jnp.int32),
            pltpu.VMEM((_EC,), jnp.int32),
            pltpu.VMEM((_EC,), jnp.float32),
            pltpu.VMEM((_EC,), jnp.int32),
            pltpu.VMEM((_EC,), jnp.int32),
            pltpu.VMEM((_EC,), jnp.float32),
            pltpu.VMEM((_EC,), jnp.float32),
            pltpu.VMEM((_ZRF,), jnp.float32),
            pltpu.VMEM_SHARED((2 * N_NODES,), jnp.float32),
            pltpu.SemaphoreType.DMA,
            pltpu.SemaphoreType.DMA,
        ],
    )
    def deg(dst_hbm, w_hbm, out_hbm, dall, wall, de0, do0, wc0, de1, do1,
            wc1, one_v, z_v, shared, sem0, sem1):
        cid = lax.axis_index("c")
        sid = lax.axis_index("s")
        wid = sid * _NC + cid
        de = (de0, de1)
        do = (do0, do1)
        wc = (wc0, wc1)
        sems = (sem0, sem1)

        z16 = jnp.zeros((16,), jnp.float32)
        o16 = jnp.ones((16,), jnp.float32)

        def zi(i, _):
            z_v[pl.ds(i * 16, 16)] = z16
            return 0

        lax.fori_loop(0, _ZRF // 16, zi, 0)
        for g in range(_EC // 16):
            one_v[pl.ds(g * 16, 16)] = o16

        @pl.when(sid < _CPT)
        def _():
            zb = pl.multiple_of(sid * _ZRF, 8)
            pltpu.sync_copy(z_v, shared.at[pl.ds(zb, _ZRF)])

        base = pl.multiple_of(wid * _EPW, 8)
        pltpu.sync_copy(dst_hbm.at[pl.ds(base, _EPW)], dall)
        pltpu.sync_copy(w_hbm.at[pl.ds(base, _EPW)], wall)
        plsc.subcore_barrier()

        def issue(ch, p):
            for g in range(_EC // 16):
                sl = pl.ds(g * 16, 16)
                esl = pl.ds(ch * _EC + g * 16, 16)
                d2 = dall[esl] * 2
                de[p][sl] = d2
                do[p][sl] = d2 + 1
                wc[p][sl] = wall[esl]
            pltpu.async_copy(wc[p], shared.at[de[p]], sems[p], add=True)
            pltpu.async_copy(one_v, shared.at[do[p]], sems[p], add=True)

        def drain(p):
            pltpu.make_async_copy(wc[p], shared.at[de[p]], sems[p]).wait()
            pltpu.make_async_copy(one_v, shared.at[do[p]], sems[p]).wait()

        def do_deg(ch, par):
            @pl.when(ch >= 2)
            def _():
                drain(par)

            issue(ch, par)

        def pair(i, _):
            do_deg(2 * i, 0)
            do_deg(2 * i + 1, 1)
            return 0

        lax.fori_loop(0, _NCH // 2, pair, 0)
        do_deg(_NCH - 1, 0)
        drain(1)
        drain(0)
        plsc.subcore_barrier()

        @pl.when(sid < _CPT)
        def _():
            zb = pl.multiple_of(sid * _ZRF, 8)
            ob = pl.multiple_of(cid * 2 * N_NODES + sid * _ZRF, 8)
            pltpu.sync_copy(shared.at[pl.ds(zb, _ZRF)], z_v)
            pltpu.sync_copy(z_v, out_hbm.at[pl.ds(ob, _ZRF)])

    return deg(dst, w).reshape(_NC, N_NODES, 2)


def _agg(src, dst, w, rows, weighted):
    @functools.partial(
        pl.kernel,
        out_type=jax.ShapeDtypeStruct((_NC, N_NODES, H), jnp.float32),
        mesh=_sc_mesh(),
        scratch_types=[
            pltpu.VMEM((_EPW,), jnp.int32),
            pltpu.VMEM((_EPW,), jnp.int32),
            pltpu.VMEM((_EPW,), jnp.float32),
            pltpu.VMEM((_EC,), jnp.int32),
            pltpu.VMEM((_EC,), jnp.int32),
            pltpu.VMEM((_EC,), jnp.int32),
            pltpu.VMEM((_EC,), jnp.int32),
            pltpu.VMEM((_EC, H), jnp.float32),
            pltpu.VMEM((_EC, H), jnp.float32),
            pltpu.VMEM((_ZR, H), jnp.float32),
            pltpu.VMEM_SHARED((N_NODES, H), jnp.float32),
            pltpu.SemaphoreType.DMA,
            pltpu.SemaphoreType.DMA,
            pltpu.SemaphoreType.DMA,
            pltpu.SemaphoreType.DMA,
        ],
        compiler_params=pltpu.CompilerParams(use_tc_tiling_on_sc=False),
    )
    def agg(src_hbm, dst_hbm, w_hbm, rows_hbm, out_hbm,
            sall, dall, wall, si0, si1, di0, di1, rb0, rb1, z_v, shared,
            gs0, gs1, ss0, ss1):
        cid = lax.axis_index("c")
        sid = lax.axis_index("s")
        wid = sid * _NC + cid
        si = (si0, si1)
        di = (di0, di1)
        rb = (rb0, rb1)
        gs = (gs0, gs1)
        ss = (ss0, ss1)

        z16 = jnp.zeros((16,), jnp.float32)

        def zi(i, _):
            for q in range(H // 16):
                z_v[i, pl.ds(q * 16, 16)] = z16
            return 0

        lax.fori_loop(0, _ZR, zi, 0)

        @pl.when(sid < _CPT)
        def _():
            for j in range(_RPT // _ZR):
                r0 = pl.multiple_of(sid * _RPT + j * _ZR, 8)
                pltpu.sync_copy(z_v, shared.at[pl.ds(r0, _ZR)])

        base = pl.multiple_of(wid * _EPW, 8)
        pltpu.sync_copy(src_hbm.at[pl.ds(base, _EPW)], sall)
        pltpu.sync_copy(dst_hbm.at[pl.ds(base, _EPW)], dall)
        if weighted:
            pltpu.sync_copy(w_hbm.at[pl.ds(base, _EPW)], wall)
        plsc.subcore_barrier()

        def start_gather(ch, p):
            for g in range(_EC // 16):
                sl = pl.ds(g * 16, 16)
                esl = pl.ds(ch * _EC + g * 16, 16)
                si[p][sl] = sall[esl]
                di[p][sl] = dall[esl]
            pltpu.async_copy(rows_hbm.at[si[p]], rb[p], gs[p])

        start_gather(0, 0)

        def do_chunk(ch, par):
            q = 1 - par

            @pl.when(ch + 1 < _NCH)
            def _():
                @pl.when(ch >= 1)
                def _():
                    pltpu.make_async_copy(
                        rb[q], shared.at[di[q]], ss[q]).wait()

                start_gather(ch + 1, q)

            pltpu.make_async_copy(rows_hbm.at[si[par]], rb[par],
                                  gs[par]).wait()
            if weighted:
                for g in range(_EC // 16):
                    w16 = wall[pl.ds(ch * _EC + g * 16, 16)]
                    for e in range(16):
                        sp = w16.at[jnp.full((16,), e, jnp.int32)].get(
                            mode="promise_in_bounds")
                        r = g * 16 + e
                        for qq in range(H // 16):
                            sl = pl.ds(qq * 16, 16)
                            rb[par][r, sl] = rb[par][r, sl] * sp
            pltpu.async_copy(rb[par], shared.at[di[par]], ss[par], add=True)

        def pair(i, _):
            do_chunk(2 * i, 0)
            do_chunk(2 * i + 1, 1)
            return 0

        lax.fori_loop(0, _NCH // 2, pair, 0)
        do_chunk(_NCH - 1, 0)
        pltpu.make_async_copy(rb[1], shared.at[di[1]], ss[1]).wait()
        pltpu.make_async_copy(rb[0], shared.at[di[0]], ss[0]).wait()
        plsc.subcore_barrier()

        @pl.when(sid < _CPT)
        def _():
            for j in range(_RPT // _ZR):
                r0 = pl.multiple_of(sid * _RPT + j * _ZR, 8)
                pltpu.sync_copy(shared.at[pl.ds(r0, _ZR)], z_v)
                pltpu.sync_copy(z_v, out_hbm.at[cid, pl.ds(r0, _ZR)])

    a = agg(src, dst, w, rows)
    return a[0], a[1]


def kernel(x, edge_index, edge_weight, W1, b1, W2, b2):
    src = edge_index[0]
    dst = edge_index[1]
    degp = _deg_partials(dst, edge_weight)

    hs1, dinv = _prep_call(x, W1, degp)
    dinv1 = dinv[:, 0:1]
    dinv2 = dinv[:, 1:2]

    a0, a1 = _agg(src, dst, edge_weight, hs1, True)
    hs2 = _layer_call(a0, a1, hs1, dinv1, b1.reshape(1, H), W2, dinv2)

    b0p, b1p = _agg(src, dst, edge_weight, hs2, False)
    out = _layer_call(b0p, b1p, hs2, dinv2, b2.reshape(1, H))
    return out


# speedup vs baseline: 31.6021x; 1.0320x over previous
import functools

import jax
import jax.numpy as jnp
from jax.experimental import pallas as pl
from jax.experimental.pallas import tpu as pltpu

N_NODES = 10000
N_EDGES = 320000
D_IN = 128
H = 64
K = 16
KM_ITERS = 10

_INIT_IDX = (0, 666, 1333, 1999, 2666, 3333, 3999, 4666,
             5332, 5999, 6666, 7332, 7999, 8665, 9332, 9999)

_INTERPRET = False


_BLK = 500
_NBLK = N_NODES // _BLK
_BLKS = 5000
_NBLKS = N_NODES // _BLKS
_BLKF = 2500
_NBLKF = N_NODES // _BLKF


def _prep_body(x_ref, w1_ref, degs_ref, hs_ref, dinv_ref):
    w1 = w1_ref[...]

    def blk(i, _):
        rows = pl.ds(i * _BLK, _BLK)
        s = degs_ref[0, rows, :] + degs_ref[1, rows, :]
        dinv = jax.lax.rsqrt(1.0 + s)
        h = jnp.dot(x_ref[rows, :], w1, preferred_element_type=jnp.float32)
        hs_ref[rows, :] = h * dinv[:, 0:1]
        dinv_ref[rows, :] = dinv
        return 0

    jax.lax.fori_loop(0, _NBLK, blk, 0)


def _prep_call(x, w1, deg_partials):
    return pl.pallas_call(
        _prep_body,
        out_shape=(
            jax.ShapeDtypeStruct((N_NODES, H), jnp.float32),
            jax.ShapeDtypeStruct((N_NODES, 2), jnp.float32),
        ),
        interpret=_INTERPRET,
    )(x, w1, deg_partials)


def _layer_body(final, *refs):
    if final:
        a0_ref, a1_ref, hs_ref, dinv_ref, b_ref, out_ref, h_ref = refs
        wn_ref = dinvn_ref = None
    else:
        (a0_ref, a1_ref, hs_ref, dinv_ref, b_ref, wn_ref, dinvn_ref,
         out_ref, h_ref) = refs
    b = b_ref[...]

    def hblk(i, _):
        rows = pl.ds(i * _BLKF, _BLKF)
        h_ref[rows, :] = ((a0_ref[rows, :] + a1_ref[rows, :]
                           + hs_ref[rows, :]) * dinv_ref[rows, :] + b)
        return 0

    jax.lax.fori_loop(0, _NBLKF, hblk, 0)

    c0 = jnp.concatenate([h_ref[i:i + 1, :] for i in _INIT_IDX], axis=0)

    ones_b = jnp.ones((_BLKS, 1), jnp.float32)

    def stats(c, with_sq):
        cn = jnp.sum(c * c, axis=1)

        def blk(i, carry):
            counts, sums, sqs = carry
            rows = pl.ds(i * _BLKS, _BLKS)
            hb = h_ref[rows, :]
            g = jax.lax.dot_general(c, hb, (((1,), (1,)), ((), ())),
                                    preferred_element_type=jnp.float32,
                                    precision=jax.lax.Precision.HIGHEST)
            s = cn[:, None] - 2.0 * g
            mn = jnp.min(s, axis=0, keepdims=True)
            rowi = jax.lax.broadcasted_iota(
                jnp.int32, (K, _BLKS), 0).astype(jnp.float32)
            masked = jnp.where(s == mn, rowi, jnp.float32(K))
            amin = jnp.min(masked, axis=0, keepdims=True)
            oh = (rowi == amin).astype(jnp.float32)
            counts = counts + jnp.dot(
                oh, ones_b, preferred_element_type=jnp.float32,
                precision=jax.lax.Precision.HIGHEST)
            sums = sums + jnp.dot(oh, hb, preferred_element_type=jnp.float32,
                                  precision=jax.lax.Precision.HIGHEST)
            if with_sq:
                sqs = sqs + jnp.dot(oh, hb * hb,
                                    preferred_element_type=jnp.float32,
                                    precision=jax.lax.Precision.HIGHEST)
            return counts, sums, sqs

        z = jnp.zeros((K, H), jnp.float32)
        init = (jnp.zeros((K, 1), jnp.float32), z, z)
        counts, sums, sqs = jax.lax.fori_loop(0, _NBLKS, blk, init)
        return counts.reshape(K), sums, sqs

    def iter_fn(t, c):
        counts, sums, _ = stats(c, False)
        newc = sums / jnp.maximum(counts, 1.0)[:, None]
        return jnp.where(counts[:, None] > 0, newc, c)

    c = jax.lax.fori_loop(0, KM_ITERS, iter_fn, c0)

    counts, _, sqs = stats(c, True)
    counts = jnp.maximum(counts, 1.0)
    sq = sqs / counts[:, None] - c * c
    vs = jnp.sqrt(jnp.maximum(sq, 1e-4))
    p = 0.5 / (vs * vs)

    const = jnp.sum(c * c * p, axis=1)
    cp2 = 2.0 * c * p
    wn = None if final else wn_ref[...]

    def fblk(i, _):
        rows = pl.ds(i * _BLKF, _BLKF)
        hb = h_ref[rows, :]
        hhb = hb * hb
        t1 = jax.lax.dot_general(p, hhb, (((1,), (1,)), ((), ())),
                                 preferred_element_type=jnp.float32,
                                 precision=jax.lax.Precision.HIGHEST)
        t2 = jax.lax.dot_general(cp2, hb, (((1,), (1,)), ((), ())),
                                 preferred_element_type=jnp.float32,
                                 precision=jax.lax.Precision.HIGHEST)
        logm = t2 - t1 - const[:, None]
        m = jnp.max(logm, axis=0, keepdims=True)
        e = jnp.exp(logm - m)
        frs = e / jnp.sum(e, axis=0, keepdims=True)
        fuzz = jax.lax.dot_general(frs, c, (((0,), (0,)), ((), ())),
                                   preferred_element_type=jnp.float32)
        if final:
            out_ref[rows, :] = fuzz
        else:
            out_ref[rows, :] = (jnp.dot(fuzz, wn,
                                        preferred_element_type=jnp.float32)
                                * dinvn_ref[rows, :])
        return 0

    jax.lax.fori_loop(0, _NBLKF, fblk, 0)


def _layer_call(a0, a1, hs, dinv, b, wn=None, dinvn=None):
    final = wn is None
    args = (a0, a1, hs, dinv, b) if final else (a0, a1, hs, dinv, b, wn,
                                                dinvn)
    return pl.pallas_call(
        functools.partial(_layer_body, final),
        out_shape=jax.ShapeDtypeStruct((N_NODES, H), jnp.float32),
        scratch_shapes=[pltpu.VMEM((N_NODES, H), jnp.float32)],
        interpret=_INTERPRET,
    )(*args)


from jax import lax
from jax.experimental.pallas import tpu_sc as plsc

_NC = 2
_NS = 16
_NW = _NC * _NS
_EPW = N_EDGES // _NW
_EC = 80
_NCH = _EPW // _EC
_CPT = 10
_RPT = N_NODES // _CPT
_ZR = 200
_ZRF = 2 * N_NODES // _CPT


def _sc_mesh():
    return plsc.VectorSubcoreMesh(core_axis_name="c", subcore_axis_name="s",
                                  num_cores=_NC, num_subcores=_NS)


def _deg_partials(dst, w):
    @functools.partial(
        pl.kernel,
        out_type=jax.ShapeDtypeStruct((_NC * 2 * N_NODES,), jnp.float32),
        mesh=_sc_mesh(),
        scratch_types=[
            pltpu.VMEM((_EPW,), jnp.int32),
            pltpu.VMEM((_EPW,), jnp.float32),
            pltpu.VMEM((_EC,), jnp.int32),
            pltpu.VMEM((_EC,), jnp.int32),
            pltpu.VMEM((_EC,), jnp.float32),
            pltpu.VMEM((_EC,), jnp.int32),
            pltpu.VMEM((_EC,), jnp.int32),
            pltpu.VMEM((_EC,), jnp.float32),
            pltpu.VMEM((_EC,), jnp.float32),
            pltpu.VMEM((_ZRF,), jnp.float32),
            pltpu.VMEM_SHARED((2 * N_NODES,), jnp.float32),
            pltpu.SemaphoreType.DMA,
            pltpu.SemaphoreType.DMA,
        ],
    )
    def deg(dst_hbm, w_hbm, out_hbm, dall, wall, de0, do0, wc0, de1, do1,
            wc1, one_v, z_v, shared, sem0, sem1):
        cid = lax.axis_index("c")
        sid = lax.axis_index("s")
        wid = sid * _NC + cid
        de = (de0, de1)
        do = (do0, do1)
        wc = (wc0, wc1)
        sems = (sem0, sem1)

        z16 = jnp.zeros((16,), jnp.float32)
        o16 = jnp.ones((16,), jnp.float32)

        def zi(i, _):
            z_v[pl.ds(i * 16, 16)] = z16
            return 0

        lax.fori_loop(0, _ZRF // 16, zi, 0)
        for g in range(_EC // 16):
            one_v[pl.ds(g * 16, 16)] = o16

        @pl.when(sid < _CPT)
        def _():
            zb = pl.multiple_of(sid * _ZRF, 8)
            pltpu.sync_copy(z_v, shared.at[pl.ds(zb, _ZRF)])

        base = pl.multiple_of(wid * _EPW, 8)
        pltpu.sync_copy(dst_hbm.at[pl.ds(base, _EPW)], dall)
        pltpu.sync_copy(w_hbm.at[pl.ds(base, _EPW)], wall)
        plsc.subcore_barrier()

        def issue(ch, p):
            for g in range(_EC // 16):
                sl = pl.ds(g * 16, 16)
                esl = pl.ds(ch * _EC + g * 16, 16)
                d2 = dall[esl] * 2
                de[p][sl] = d2
                do[p][sl] = d2 + 1
                wc[p][sl] = wall[esl]
            pltpu.async_copy(wc[p], shared.at[de[p]], sems[p], add=True)
            pltpu.async_copy(one_v, shared.at[do[p]], sems[p], add=True)

        def drain(p):
            pltpu.make_async_copy(wc[p], shared.at[de[p]], sems[p]).wait()
            pltpu.make_async_copy(one_v, shared.at[do[p]], sems[p]).wait()

        def do_deg(ch, par):
            @pl.when(ch >= 2)
            def _():
                drain(par)

            issue(ch, par)

        def pair(i, _):
            do_deg(2 * i, 0)
            do_deg(2 * i + 1, 1)
            return 0

        lax.fori_loop(0, _NCH // 2, pair, 0)
        do_deg(_NCH - 1, 0)
        drain(1)
        drain(0)
        plsc.subcore_barrier()

        @pl.when(sid < _CPT)
        def _():
            zb = pl.multiple_of(sid * _ZRF, 8)
            ob = pl.multiple_of(cid * 2 * N_NODES + sid * _ZRF, 8)
            pltpu.sync_copy(shared.at[pl.ds(zb, _ZRF)], z_v)
            pltpu.sync_copy(z_v, out_hbm.at[pl.ds(ob, _ZRF)])

    return deg(dst, w).reshape(_NC, N_NODES, 2)


def _agg(src, dst, w, rows, weighted):
    @functools.partial(
        pl.kernel,
        out_type=jax.ShapeDtypeStruct((_NC, N_NODES, H), jnp.float32),
        mesh=_sc_mesh(),
        scratch_types=[
            pltpu.VMEM((_EPW,), jnp.int32),
            pltpu.VMEM((_EPW,), jnp.int32),
            pltpu.VMEM((_EPW,), jnp.float32),
            pltpu.VMEM((_EC,), jnp.int32),
            pltpu.VMEM((_EC,), jnp.int32),
            pltpu.VMEM((_EC,), jnp.int32),
            pltpu.VMEM((_EC,), jnp.int32),
            pltpu.VMEM((_EC, H), jnp.float32),
            pltpu.VMEM((_EC, H), jnp.float32),
            pltpu.VMEM((_ZR, H), jnp.float32),
            pltpu.VMEM_SHARED((N_NODES, H), jnp.float32),
            pltpu.SemaphoreType.DMA,
            pltpu.SemaphoreType.DMA,
            pltpu.SemaphoreType.DMA,
            pltpu.SemaphoreType.DMA,
        ],
        compiler_params=pltpu.CompilerParams(use_tc_tiling_on_sc=False),
    )
    def agg(src_hbm, dst_hbm, w_hbm, rows_hbm, out_hbm,
            sall, dall, wall, si0, si1, di0, di1, rb0, rb1, z_v, shared,
            gs0, gs1, ss0, ss1):
        cid = lax.axis_index("c")
        sid = lax.axis_index("s")
        wid = sid * _NC + cid
        si = (si0, si1)
        di = (di0, di1)
        rb = (rb0, rb1)
        gs = (gs0, gs1)
        ss = (ss0, ss1)

        z16 = jnp.zeros((16,), jnp.float32)

        def zi(i, _):
            for q in range(H // 16):
                z_v[i, pl.ds(q * 16, 16)] = z16
            return 0

        lax.fori_loop(0, _ZR, zi, 0)

        @pl.when(sid < _CPT)
        def _():
            for j in range(_RPT // _ZR):
                r0 = pl.multiple_of(sid * _RPT + j * _ZR, 8)
                pltpu.sync_copy(z_v, shared.at[pl.ds(r0, _ZR)])

        base = pl.multiple_of(wid * _EPW, 8)
        pltpu.sync_copy(src_hbm.at[pl.ds(base, _EPW)], sall)
        pltpu.sync_copy(dst_hbm.at[pl.ds(base, _EPW)], dall)
        if weighted:
            pltpu.sync_copy(w_hbm.at[pl.ds(base, _EPW)], wall)
        plsc.subcore_barrier()

        def start_gather(ch, p):
            for g in range(_EC // 16):
                sl = pl.ds(g * 16, 16)
                esl = pl.ds(ch * _EC + g * 16, 16)
                si[p][sl] = sall[esl]
                di[p][sl] = dall[esl]
            pltpu.async_copy(rows_hbm.at[si[p]], rb[p], gs[p])

        start_gather(0, 0)

        def do_chunk(ch, par):
            q = 1 - par

            @pl.when(ch + 1 < _NCH)
            def _():
                @pl.when(ch >= 1)
                def _():
                    pltpu.make_async_copy(
                        rb[q], shared.at[di[q]], ss[q]).wait()

                start_gather(ch + 1, q)

            pltpu.make_async_copy(rows_hbm.at[si[par]], rb[par],
                                  gs[par]).wait()
            if weighted:
                for g in range(_EC // 16):
                    w16 = wall[pl.ds(ch * _EC + g * 16, 16)]
                    for e in range(16):
                        sp = w16.at[jnp.full((16,), e, jnp.int32)].get(
                            mode="promise_in_bounds")
                        r = g * 16 + e
                        for qq in range(H // 16):
                            sl = pl.ds(qq * 16, 16)
                            rb[par][r, sl] = rb[par][r, sl] * sp
            pltpu.async_copy(rb[par], shared.at[di[par]], ss[par], add=True)

        def pair(i, _):
            do_chunk(2 * i, 0)
            do_chunk(2 * i + 1, 1)
            return 0

        lax.fori_loop(0, _NCH // 2, pair, 0)
        do_chunk(_NCH - 1, 0)
        pltpu.make_async_copy(rb[1], shared.at[di[1]], ss[1]).wait()
        pltpu.make_async_copy(rb[0], shared.at[di[0]], ss[0]).wait()
        plsc.subcore_barrier()

        @pl.when(sid < _CPT)
        def _():
            for j in range(_RPT // _ZR):
                r0 = pl.multiple_of(sid * _RPT + j * _ZR, 8)
                pltpu.sync_copy(shared.at[pl.ds(r0, _ZR)], z_v)
                pltpu.sync_copy(z_v, out_hbm.at[cid, pl.ds(r0, _ZR)])

    a = agg(src, dst, w, rows)
    return a[0], a[1]


def kernel(x, edge_index, edge_weight, W1, b1, W2, b2):
    src = edge_index[0]
    dst = edge_index[1]
    degp = _deg_partials(dst, edge_weight)

    hs1, dinv = _prep_call(x, W1, degp)
    dinv1 = dinv[:, 0:1]
    dinv2 = dinv[:, 1:2]

    a0, a1 = _agg(src, dst, edge_weight, hs1, True)
    hs2 = _layer_call(a0, a1, hs1, dinv1, b1.reshape(1, H), W2, dinv2)

    b0p, b1p = _agg(src, dst, edge_weight, hs2, False)
    out = _layer_call(b0p, b1p, hs2, dinv2, b2.reshape(1, H))
    return out


# speedup vs baseline: 32.4583x; 1.0271x over previous
import functools

import jax
import jax.numpy as jnp
from jax.experimental import pallas as pl
from jax.experimental.pallas import tpu as pltpu

N_NODES = 10000
N_EDGES = 320000
D_IN = 128
H = 64
K = 16
KM_ITERS = 10

_INIT_IDX = (0, 666, 1333, 1999, 2666, 3333, 3999, 4666,
             5332, 5999, 6666, 7332, 7999, 8665, 9332, 9999)

_INTERPRET = False


_BLK = 500
_NBLK = N_NODES // _BLK
_BLKS = 10000
_NBLKS = N_NODES // _BLKS
_BLKF = 5000
_NBLKF = N_NODES // _BLKF


def _prep_body(x_ref, w1_ref, degs_ref, hs_ref, dinv_ref):
    w1 = w1_ref[...]

    def blk(i, _):
        rows = pl.ds(i * _BLK, _BLK)
        s = degs_ref[0, rows, :] + degs_ref[1, rows, :]
        dinv = jax.lax.rsqrt(1.0 + s)
        h = jnp.dot(x_ref[rows, :], w1, preferred_element_type=jnp.float32)
        hs_ref[rows, :] = h * dinv[:, 0:1]
        dinv_ref[rows, :] = dinv
        return 0

    jax.lax.fori_loop(0, _NBLK, blk, 0)


def _prep_call(x, w1, deg_partials):
    return pl.pallas_call(
        _prep_body,
        out_shape=(
            jax.ShapeDtypeStruct((N_NODES, H), jnp.float32),
            jax.ShapeDtypeStruct((N_NODES, 2), jnp.float32),
        ),
        interpret=_INTERPRET,
    )(x, w1, deg_partials)


def _layer_body(final, *refs):
    if final:
        a0_ref, a1_ref, hs_ref, dinv_ref, b_ref, out_ref, h_ref = refs
        wn_ref = dinvn_ref = None
    else:
        (a0_ref, a1_ref, hs_ref, dinv_ref, b_ref, wn_ref, dinvn_ref,
         out_ref, h_ref) = refs
    b = b_ref[...]

    def hblk(i, _):
        rows = pl.ds(i * _BLKF, _BLKF)
        h_ref[rows, :] = ((a0_ref[rows, :] + a1_ref[rows, :]
                           + hs_ref[rows, :]) * dinv_ref[rows, :] + b)
        return 0

    jax.lax.fori_loop(0, _NBLKF, hblk, 0)

    c0 = jnp.concatenate([h_ref[i:i + 1, :] for i in _INIT_IDX], axis=0)

    ones_b = jnp.ones((_BLKS, 1), jnp.float32)

    def stats(c, with_sq):
        cn = jnp.sum(c * c, axis=1)

        def blk(i, carry):
            counts, sums, sqs = carry
            rows = pl.ds(i * _BLKS, _BLKS)
            hb = h_ref[rows, :]
            g = jax.lax.dot_general(c, hb, (((1,), (1,)), ((), ())),
                                    preferred_element_type=jnp.float32,
                                    precision=jax.lax.Precision.HIGHEST)
            s = cn[:, None] - 2.0 * g
            mn = jnp.min(s, axis=0, keepdims=True)
            rowi = jax.lax.broadcasted_iota(
                jnp.int32, (K, _BLKS), 0).astype(jnp.float32)
            masked = jnp.where(s == mn, rowi, jnp.float32(K))
            amin = jnp.min(masked, axis=0, keepdims=True)
            oh = (rowi == amin).astype(jnp.float32)
            counts = counts + jnp.dot(
                oh, ones_b, preferred_element_type=jnp.float32,
                precision=jax.lax.Precision.HIGHEST)
            sums = sums + jnp.dot(oh, hb, preferred_element_type=jnp.float32,
                                  precision=jax.lax.Precision.HIGHEST)
            if with_sq:
                sqs = sqs + jnp.dot(oh, hb * hb,
                                    preferred_element_type=jnp.float32,
                                    precision=jax.lax.Precision.HIGHEST)
            return counts, sums, sqs

        z = jnp.zeros((K, H), jnp.float32)
        init = (jnp.zeros((K, 1), jnp.float32), z, z)
        counts, sums, sqs = jax.lax.fori_loop(0, _NBLKS, blk, init)
        return counts.reshape(K), sums, sqs

    def iter_fn(t, c):
        counts, sums, _ = stats(c, False)
        newc = sums / jnp.maximum(counts, 1.0)[:, None]
        return jnp.where(counts[:, None] > 0, newc, c)

    c = jax.lax.fori_loop(0, KM_ITERS, iter_fn, c0)

    counts, _, sqs = stats(c, True)
    counts = jnp.maximum(counts, 1.0)
    sq = sqs / counts[:, None] - c * c
    vs = jnp.sqrt(jnp.maximum(sq, 1e-4))
    p = 0.5 / (vs * vs)

    const = jnp.sum(c * c * p, axis=1)
    cp2 = 2.0 * c * p
    wn = None if final else wn_ref[...]

    def fblk(i, _):
        rows = pl.ds(i * _BLKF, _BLKF)
        hb = h_ref[rows, :]
        hhb = hb * hb
        t1 = jax.lax.dot_general(p, hhb, (((1,), (1,)), ((), ())),
                                 preferred_element_type=jnp.float32,
                                 precision=jax.lax.Precision.HIGHEST)
        t2 = jax.lax.dot_general(cp2, hb, (((1,), (1,)), ((), ())),
                                 preferred_element_type=jnp.float32,
                                 precision=jax.lax.Precision.HIGHEST)
        logm = t2 - t1 - const[:, None]
        m = jnp.max(logm, axis=0, keepdims=True)
        e = jnp.exp(logm - m)
        frs = e / jnp.sum(e, axis=0, keepdims=True)
        fuzz = jax.lax.dot_general(frs, c, (((0,), (0,)), ((), ())),
                                   preferred_element_type=jnp.float32)
        if final:
            out_ref[rows, :] = fuzz
        else:
            out_ref[rows, :] = (jnp.dot(fuzz, wn,
                                        preferred_element_type=jnp.float32)
                                * dinvn_ref[rows, :])
        return 0

    jax.lax.fori_loop(0, _NBLKF, fblk, 0)


def _layer_call(a0, a1, hs, dinv, b, wn=None, dinvn=None):
    final = wn is None
    args = (a0, a1, hs, dinv, b) if final else (a0, a1, hs, dinv, b, wn,
                                                dinvn)
    return pl.pallas_call(
        functools.partial(_layer_body, final),
        out_shape=jax.ShapeDtypeStruct((N_NODES, H), jnp.float32),
        scratch_shapes=[pltpu.VMEM((N_NODES, H), jnp.float32)],
        interpret=_INTERPRET,
    )(*args)


from jax import lax
from jax.experimental.pallas import tpu_sc as plsc

_NC = 2
_NS = 16
_NW = _NC * _NS
_EPW = N_EDGES // _NW
_EC = 80
_NCH = _EPW // _EC
_CPT = 10
_RPT = N_NODES // _CPT
_ZR = 200
_ZRF = 2 * N_NODES // _CPT


def _sc_mesh():
    return plsc.VectorSubcoreMesh(core_axis_name="c", subcore_axis_name="s",
                                  num_cores=_NC, num_subcores=_NS)


def _deg_partials(dst, w):
    @functools.partial(
        pl.kernel,
        out_type=jax.ShapeDtypeStruct((_NC * 2 * N_NODES,), jnp.float32),
        mesh=_sc_mesh(),
        scratch_types=[
            pltpu.VMEM((_EPW,), jnp.int32),
            pltpu.VMEM((_EPW,), jnp.float32),
            pltpu.VMEM((_EC,), jnp.int32),
            pltpu.VMEM((_EC,), jnp.int32),
            pltpu.VMEM((_EC,), jnp.float32),
            pltpu.VMEM((_EC,), jnp.int32),
            pltpu.VMEM((_EC,), jnp.int32),
            pltpu.VMEM((_EC,), jnp.float32),
            pltpu.VMEM((_EC,), jnp.float32),
            pltpu.VMEM((_ZRF,), jnp.float32),
            pltpu.VMEM_SHARED((2 * N_NODES,), jnp.float32),
            pltpu.SemaphoreType.DMA,
            pltpu.SemaphoreType.DMA,
        ],
    )
    def deg(dst_hbm, w_hbm, out_hbm, dall, wall, de0, do0, wc0, de1, do1,
            wc1, one_v, z_v, shared, sem0, sem1):
        cid = lax.axis_index("c")
        sid = lax.axis_index("s")
        wid = sid * _NC + cid
        de = (de0, de1)
        do = (do0, do1)
        wc = (wc0, wc1)
        sems = (sem0, sem1)

        z16 = jnp.zeros((16,), jnp.float32)
        o16 = jnp.ones((16,), jnp.float32)

        def zi(i, _):
            z_v[pl.ds(i * 16, 16)] = z16
            return 0

        lax.fori_loop(0, _ZRF // 16, zi, 0)
        for g in range(_EC // 16):
            one_v[pl.ds(g * 16, 16)] = o16

        @pl.when(sid < _CPT)
        def _():
            zb = pl.multiple_of(sid * _ZRF, 8)
            pltpu.sync_copy(z_v, shared.at[pl.ds(zb, _ZRF)])

        base = pl.multiple_of(wid * _EPW, 8)
        pltpu.sync_copy(dst_hbm.at[pl.ds(base, _EPW)], dall)
        pltpu.sync_copy(w_hbm.at[pl.ds(base, _EPW)], wall)
        plsc.subcore_barrier()

        def issue(ch, p):
            for g in range(_EC // 16):
                sl = pl.ds(g * 16, 16)
                esl = pl.ds(ch * _EC + g * 16, 16)
                d2 = dall[esl] * 2
                de[p][sl] = d2
                do[p][sl] = d2 + 1
                wc[p][sl] = wall[esl]
            pltpu.async_copy(wc[p], shared.at[de[p]], sems[p], add=True)
            pltpu.async_copy(one_v, shared.at[do[p]], sems[p], add=True)

        def drain(p):
            pltpu.make_async_copy(wc[p], shared.at[de[p]], sems[p]).wait()
            pltpu.make_async_copy(one_v, shared.at[do[p]], sems[p]).wait()

        def do_deg(ch, par):
            @pl.when(ch >= 2)
            def _():
                drain(par)

            issue(ch, par)

        def pair(i, _):
            do_deg(2 * i, 0)
            do_deg(2 * i + 1, 1)
            return 0

        lax.fori_loop(0, _NCH // 2, pair, 0)
        do_deg(_NCH - 1, 0)
        drain(1)
        drain(0)
        plsc.subcore_barrier()

        @pl.when(sid < _CPT)
        def _():
            zb = pl.multiple_of(sid * _ZRF, 8)
            ob = pl.multiple_of(cid * 2 * N_NODES + sid * _ZRF, 8)
            pltpu.sync_copy(shared.at[pl.ds(zb, _ZRF)], z_v)
            pltpu.sync_copy(z_v, out_hbm.at[pl.ds(ob, _ZRF)])

    return deg(dst, w).reshape(_NC, N_NODES, 2)


def _agg(src, dst, w, rows, weighted):
    @functools.partial(
        pl.kernel,
        out_type=jax.ShapeDtypeStruct((_NC, N_NODES, H), jnp.float32),
        mesh=_sc_mesh(),
        scratch_types=[
            pltpu.VMEM((_EPW,), jnp.int32),
            pltpu.VMEM((_EPW,), jnp.int32),
            pltpu.VMEM((_EPW,), jnp.float32),
            pltpu.VMEM((_EC,), jnp.int32),
            pltpu.VMEM((_EC,), jnp.int32),
            pltpu.VMEM((_EC,), jnp.int32),
            pltpu.VMEM((_EC,), jnp.int32),
            pltpu.VMEM((_EC, H), jnp.float32),
            pltpu.VMEM((_EC, H), jnp.float32),
            pltpu.VMEM((_ZR, H), jnp.float32),
            pltpu.VMEM_SHARED((N_NODES, H), jnp.float32),
            pltpu.SemaphoreType.DMA,
            pltpu.SemaphoreType.DMA,
            pltpu.SemaphoreType.DMA,
            pltpu.SemaphoreType.DMA,
        ],
        compiler_params=pltpu.CompilerParams(use_tc_tiling_on_sc=False),
    )
    def agg(src_hbm, dst_hbm, w_hbm, rows_hbm, out_hbm,
            sall, dall, wall, si0, si1, di0, di1, rb0, rb1, z_v, shared,
            gs0, gs1, ss0, ss1):
        cid = lax.axis_index("c")
        sid = lax.axis_index("s")
        wid = sid * _NC + cid
        si = (si0, si1)
        di = (di0, di1)
        rb = (rb0, rb1)
        gs = (gs0, gs1)
        ss = (ss0, ss1)

        z16 = jnp.zeros((16,), jnp.float32)

        def zi(i, _):
            for q in range(H // 16):
                z_v[i, pl.ds(q * 16, 16)] = z16
            return 0

        lax.fori_loop(0, _ZR, zi, 0)

        @pl.when(sid < _CPT)
        def _():
            for j in range(_RPT // _ZR):
                r0 = pl.multiple_of(sid * _RPT + j * _ZR, 8)
                pltpu.sync_copy(z_v, shared.at[pl.ds(r0, _ZR)])

        base = pl.multiple_of(wid * _EPW, 8)
        pltpu.sync_copy(src_hbm.at[pl.ds(base, _EPW)], sall)
        pltpu.sync_copy(dst_hbm.at[pl.ds(base, _EPW)], dall)
        if weighted:
            pltpu.sync_copy(w_hbm.at[pl.ds(base, _EPW)], wall)
        plsc.subcore_barrier()

        def start_gather(ch, p):
            for g in range(_EC // 16):
                sl = pl.ds(g * 16, 16)
                esl = pl.ds(ch * _EC + g * 16, 16)
                si[p][sl] = sall[esl]
                di[p][sl] = dall[esl]
            pltpu.async_copy(rows_hbm.at[si[p]], rb[p], gs[p])

        start_gather(0, 0)

        def do_chunk(ch, par):
            q = 1 - par

            @pl.when(ch + 1 < _NCH)
            def _():
                @pl.when(ch >= 1)
                def _():
                    pltpu.make_async_copy(
                        rb[q], shared.at[di[q]], ss[q]).wait()

                start_gather(ch + 1, q)

            pltpu.make_async_copy(rows_hbm.at[si[par]], rb[par],
                                  gs[par]).wait()
            if weighted:
                for g in range(_EC // 16):
                    w16 = wall[pl.ds(ch * _EC + g * 16, 16)]
                    for e in range(16):
                        sp = w16.at[jnp.full((16,), e, jnp.int32)].get(
                            mode="promise_in_bounds")
                        r = g * 16 + e
                        for qq in range(H // 16):
                            sl = pl.ds(qq * 16, 16)
                            rb[par][r, sl] = rb[par][r, sl] * sp
            pltpu.async_copy(rb[par], shared.at[di[par]], ss[par], add=True)

        def pair(i, _):
            do_chunk(2 * i, 0)
            do_chunk(2 * i + 1, 1)
            return 0

        lax.fori_loop(0, _NCH // 2, pair, 0)
        do_chunk(_NCH - 1, 0)
        pltpu.make_async_copy(rb[1], shared.at[di[1]], ss[1]).wait()
        pltpu.make_async_copy(rb[0], shared.at[di[0]], ss[0]).wait()
        plsc.subcore_barrier()

        @pl.when(sid < _CPT)
        def _():
            for j in range(_RPT // _ZR):
                r0 = pl.multiple_of(sid * _RPT + j * _ZR, 8)
                pltpu.sync_copy(shared.at[pl.ds(r0, _ZR)], z_v)
                pltpu.sync_copy(z_v, out_hbm.at[cid, pl.ds(r0, _ZR)])

    a = agg(src, dst, w, rows)
    return a[0], a[1]


def kernel(x, edge_index, edge_weight, W1, b1, W2, b2):
    src = edge_index[0]
    dst = edge_index[1]
    degp = _deg_partials(dst, edge_weight)

    hs1, dinv = _prep_call(x, W1, degp)
    dinv1 = dinv[:, 0:1]
    dinv2 = dinv[:, 1:2]

    a0, a1 = _agg(src, dst, edge_weight, hs1, True)
    hs2 = _layer_call(a0, a1, hs1, dinv1, b1.reshape(1, H), W2, dinv2)

    b0p, b1p = _agg(src, dst, edge_weight, hs2, False)
    out = _layer_call(b0p, b1p, hs2, dinv2, b2.reshape(1, H))
    return out
